# baseline (speedup 1.0000x reference)
import functools
import math

import jax
import jax.numpy as jnp
from jax import lax
from jax.experimental import pallas as pl
from jax.experimental.pallas import tpu as pltpu
from jax.experimental.pallas import tpu_sc as plsc


def _qkv_body(x_ref, wq_ref, wk_ref, wv_ref, bq_ref, bk_ref, bv_ref,
              q_ref, k_ref, v_ref):
    x = x_ref[...]
    dims = (((1,), (1,)), ((), ()))
    q_ref[...] = lax.dot_general(x, wq_ref[...], dims,
                                 preferred_element_type=jnp.float32) + bq_ref[...]
    k_ref[...] = lax.dot_general(x, wk_ref[...], dims,
                                 preferred_element_type=jnp.float32) + bk_ref[...]
    v_ref[...] = lax.dot_general(x, wv_ref[...], dims,
                                 preferred_element_type=jnp.float32) + bv_ref[...]


def _qkv_call(x, wq, wk, wv, bq, bk, bv, rb, cb):
    n, d = x.shape
    w_spec = pl.BlockSpec((cb, d), lambda j, i: (j, 0))
    b_spec = pl.BlockSpec((1, cb), lambda j, i: (0, j))
    o_spec = pl.BlockSpec((rb, cb), lambda j, i: (i, j))
    o_shape = jax.ShapeDtypeStruct((n, d), jnp.float32)
    return pl.pallas_call(
        _qkv_body,
        grid=(d // cb, n // rb),
        in_specs=[
            pl.BlockSpec((rb, d), lambda j, i: (i, 0)),
            w_spec, w_spec, w_spec, b_spec, b_spec, b_spec,
        ],
        out_specs=[o_spec, o_spec, o_spec],
        out_shape=[o_shape, o_shape, o_shape],
    )(x, wq, wk, wv, bq, bk, bv)


def _attn_body(q_ref, k_ref, v_ref, o_ref, vprev_ref, *, rb, s_len, scale):
    sb = pl.program_id(2)
    q = q_ref[...]
    inv = 1.0 / scale
    dims = (((1,), (1,)), ((), ()))

    @pl.when(sb == 0)
    def _():
        vprev_ref[...] = jnp.zeros_like(vprev_ref)

    def blk(j):
        kj = k_ref[pl.ds(j * rb, rb), :]
        vj = v_ref[pl.ds(j * rb, rb), :]
        e = jnp.exp(lax.dot_general(q, kj, dims,
                                    preferred_element_type=jnp.float32) * inv)
        return (jnp.dot(e, vj, preferred_element_type=jnp.float32),
                jnp.sum(e, axis=-1, keepdims=True))

    def body(j, carry):
        num, den = carry
        dn, dd = blk(j)
        return (num + dn, den + dd)

    num, den = lax.fori_loop(
        0, sb, body,
        (jnp.zeros((rb, q.shape[1]), jnp.float32), jnp.zeros((rb, 1), jnp.float32)))

    kd = k_ref[pl.ds(sb * rb, rb), :]
    vd = v_ref[pl.ds(sb * rb, rb), :]
    sd = lax.dot_general(q, kd, dims, preferred_element_type=jnp.float32) * inv
    r_i = lax.broadcasted_iota(jnp.int32, (rb, rb), 0)
    c_i = lax.broadcasted_iota(jnp.int32, (rb, rb), 1)
    tril = (c_i <= r_i)
    ed = jnp.where(tril, jnp.exp(sd), 0.0)
    num = num + jnp.dot(ed, vd, preferred_element_type=jnp.float32)
    den = den + jnp.sum(ed, axis=-1, keepdims=True)

    vtot = jnp.sum(v_ref[...], axis=0, keepdims=True)
    pref_d = jnp.dot(tril.astype(jnp.float32), vd,
                     preferred_element_type=jnp.float32)
    vprefix = vprev_ref[...] + pref_d
    s_glob = sb * rb + lax.broadcasted_iota(jnp.int32, (rb, 1), 0)
    nfut = (s_len - 1 - s_glob).astype(jnp.float32)
    num = num + (vtot - vprefix)
    den = den + nfut
    vprev_ref[...] = vprev_ref[...] + jnp.sum(vd, axis=0, keepdims=True)

    o_ref[...] = num / den


def _attn_call(q_all, k_all, v_all, bdim, h, s_len, dh, rb, scale):
    n = q_all.shape[0]
    d = h * dh
    sb_n = s_len // rb
    return pl.pallas_call(
        functools.partial(_attn_body, rb=rb, s_len=s_len, scale=scale),
        grid=(bdim, h, sb_n),
        in_specs=[
            pl.BlockSpec((rb, dh), lambda b, hh, sb: (b * sb_n + sb, hh)),
            pl.BlockSpec((s_len, dh), lambda b, hh, sb: (b, hh)),
            pl.BlockSpec((s_len, dh), lambda b, hh, sb: (b, hh)),
        ],
        out_specs=pl.BlockSpec((rb, dh), lambda b, hh, sb: (b * sb_n + sb, hh)),
        out_shape=jax.ShapeDtypeStruct((n, d), jnp.float32),
        scratch_shapes=[pltpu.VMEM((1, dh), jnp.float32)],
    )(q_all, k_all, v_all)


def _router_body(x_ref, w_ref, b_ref, rp_ref, kept_ref, slot_ref, slotg_ref,
                 counts_ref, *, rb, e_num, cap, cappad, trash):
    i = pl.program_id(0)

    @pl.when(i == 0)
    def _():
        counts_ref[...] = jnp.zeros_like(counts_ref)

    logits = lax.dot_general(x_ref[...], w_ref[...], (((1,), (1,)), ((), ())),
                             preferred_element_type=jnp.float32) + b_ref[...]
    m = jnp.max(logits, axis=-1, keepdims=True)
    ex = jnp.exp(logits - m)
    denom = jnp.sum(ex, axis=-1, keepdims=True)
    probs = ex / denom
    rp = 1.0 / denom
    pm = jnp.max(probs, axis=-1, keepdims=True)
    iota_e = lax.broadcasted_iota(jnp.int32, probs.shape, 1)
    route = jnp.min(jnp.where(probs >= pm, iota_e, e_num), axis=-1,
                    keepdims=True)
    onehot = (iota_e == route).astype(jnp.float32)
    r_i = lax.broadcasted_iota(jnp.int32, (rb, rb), 0)
    c_i = lax.broadcasted_iota(jnp.int32, (rb, rb), 1)
    tril = (c_i <= r_i).astype(jnp.float32)
    csum = jnp.dot(tril, onehot, preferred_element_type=jnp.float32)
    rank_all = counts_ref[...] + csum - 1.0
    rank = jnp.sum(rank_all * onehot, axis=-1, keepdims=True)
    counts_ref[...] = counts_ref[...] + csum[rb - 1:rb, :]
    kept = rank < float(cap)
    ranki = rank.astype(jnp.int32)
    slot = route * cappad + ranki
    rp_ref[...] = rp.reshape(1, rb, 1)
    kept_ref[...] = kept.astype(jnp.float32).reshape(1, rb, 1)
    slot_ref[...] = jnp.where(kept, slot, trash).reshape(1, rb, 1)
    slotg_ref[...] = jnp.where(kept, slot, 0).reshape(1, rb, 1)


def _router_call(x2, wsw, bsw, rb, cap, cappad, trash):
    n, d = x2.shape
    e_num = wsw.shape[0]
    nb = n // rb
    outs = pl.pallas_call(
        functools.partial(_router_body, rb=rb, e_num=e_num, cap=cap,
                          cappad=cappad, trash=trash),
        grid=(nb,),
        in_specs=[
            pl.BlockSpec((rb, d), lambda i: (i, 0)),
            pl.BlockSpec((e_num, d), lambda i: (0, 0)),
            pl.BlockSpec((1, e_num), lambda i: (0, 0)),
        ],
        out_specs=[
            pl.BlockSpec((1, rb, 1), lambda i: (i, 0, 0)),
            pl.BlockSpec((1, rb, 1), lambda i: (i, 0, 0)),
            pl.BlockSpec((1, rb, 1), lambda i: (i, 0, 0)),
            pl.BlockSpec((1, rb, 1), lambda i: (i, 0, 0)),
        ],
        out_shape=[
            jax.ShapeDtypeStruct((nb, rb, 1), jnp.float32),
            jax.ShapeDtypeStruct((nb, rb, 1), jnp.float32),
            jax.ShapeDtypeStruct((nb, rb, 1), jnp.int32),
            jax.ShapeDtypeStruct((nb, rb, 1), jnp.int32),
        ],
        scratch_shapes=[pltpu.VMEM((1, e_num), jnp.float32)],
    )(x2, wsw, bsw)
    return outs


_NBUF = 3


def _chunk_pipeline(nch, rd, wr):
    reads = [None] * nch
    writes = [None] * nch
    reads[0] = rd(0)
    for c in range(nch):
        if c + 1 < nch:
            if c + 1 >= _NBUF:
                writes[c + 1 - _NBUF].wait()
            reads[c + 1] = rd(c + 1)
        reads[c].wait()
        writes[c] = wr(c)
    for c in range(max(0, nch - _NBUF), nch):
        writes[c].wait()


def _sc_dispatch(x2, slot, rows_out, d):
    n = x2.shape[0]
    info = plsc.get_sparse_core_info()
    nc, ns = info.num_cores, info.num_subcores
    nw = nc * ns
    tok_w = n // nw
    ch = 16
    nch = tok_w // ch
    mesh = plsc.VectorSubcoreMesh(core_axis_name="c", subcore_axis_name="s")

    @functools.partial(
        pl.kernel, mesh=mesh,
        out_type=jax.ShapeDtypeStruct((rows_out, d), jnp.float32),
        scratch_types=(
            [pltpu.VMEM((ch,), jnp.int32)] * nch
            + [pltpu.VMEM((_NBUF, ch, d), jnp.float32),
               pltpu.SemaphoreType.DMA,
               pltpu.SemaphoreType.DMA]
        ),
    )
    def k(x2_hbm, slot_hbm, buf_hbm, *refs):
        idx_vs = refs[:nch]
        rows_v, sem_r, sem_w = refs[nch:]
        wid = lax.axis_index("s") * nc + lax.axis_index("c")
        base = wid * tok_w
        for c in range(nch):
            pltpu.sync_copy(slot_hbm.at[pl.ds(base + c * ch, ch)], idx_vs[c])

        def rd(c):
            return pltpu.async_copy(
                x2_hbm.at[pl.ds(base + c * ch, ch)],
                rows_v.at[c % _NBUF], sem_r)

        def wr(c):
            return pltpu.async_copy(
                rows_v.at[c % _NBUF], buf_hbm.at[idx_vs[c]], sem_w)

        _chunk_pipeline(nch, rd, wr)

    return k(x2, slot)


def _sc_combine(eout, slotg, d):
    n = slotg.shape[0]
    info = plsc.get_sparse_core_info()
    nc, ns = info.num_cores, info.num_subcores
    nw = nc * ns
    tok_w = n // nw
    ch = 16
    nch = tok_w // ch
    mesh = plsc.VectorSubcoreMesh(core_axis_name="c", subcore_axis_name="s")

    @functools.partial(
        pl.kernel, mesh=mesh,
        out_type=jax.ShapeDtypeStruct((n, d), jnp.float32),
        scratch_types=(
            [pltpu.VMEM((ch,), jnp.int32)] * nch
            + [pltpu.VMEM((_NBUF, ch, d), jnp.float32),
               pltpu.SemaphoreType.DMA,
               pltpu.SemaphoreType.DMA]
        ),
    )
    def k(eout_hbm, slotg_hbm, g_hbm, *refs):
        idx_vs = refs[:nch]
        rows_v, sem_r, sem_w = refs[nch:]
        wid = lax.axis_index("s") * nc + lax.axis_index("c")
        base = wid * tok_w
        for c in range(nch):
            pltpu.sync_copy(slotg_hbm.at[pl.ds(base + c * ch, ch)], idx_vs[c])

        def rd(c):
            return pltpu.async_copy(
                eout_hbm.at[idx_vs[c]], rows_v.at[c % _NBUF], sem_r)

        def wr(c):
            return pltpu.async_copy(
                rows_v.at[c % _NBUF], g_hbm.at[pl.ds(base + c * ch, ch)], sem_w)

        _chunk_pipeline(nch, rd, wr)

    return k(eout, slotg)


def _expert_body(a_ref, w_ref, b_ref, o_ref):
    a_bf = a_ref[...].astype(jnp.bfloat16)
    w_bf = w_ref[0].astype(jnp.bfloat16)
    o_ref[...] = (
        lax.dot_general(a_bf, w_bf, (((1,), (1,)), ((), ())),
                        preferred_element_type=jnp.float32)
        + b_ref[0]
    )


def _expert_call(buf, ew, eb, cappad, rb, cb):
    e_num, d, _ = ew.shape
    ib = cappad // rb
    return pl.pallas_call(
        _expert_body,
        grid=(e_num, d // cb, ib),
        in_specs=[
            pl.BlockSpec((rb, d), lambda e, j, i: (e * ib + i, 0)),
            pl.BlockSpec((1, cb, d), lambda e, j, i: (e, j, 0)),
            pl.BlockSpec((1, 1, cb), lambda e, j, i: (e, 0, j)),
        ],
        out_specs=pl.BlockSpec((rb, cb), lambda e, j, i: (e * ib + i, j)),
        out_shape=jax.ShapeDtypeStruct((e_num * cappad, d), jnp.float32),
    )(buf, ew, eb.reshape(e_num, 1, d))


def _ln_body(g_ref, x2_ref, emb_ref, kept_ref, rp_ref, gam_ref, bet_ref, o_ref):
    kept = kept_ref[...]
    val = g_ref[...] * kept + x2_ref[...] * (1.0 - kept)
    x = val * rp_ref[...] + emb_ref[...]
    mu = jnp.mean(x, axis=-1, keepdims=True)
    xc = x - mu
    var = jnp.mean(xc * xc, axis=-1, keepdims=True)
    o_ref[...] = xc * lax.rsqrt(var + 1e-5) * gam_ref[...] + bet_ref[...]


def _ln_call(g, x2, emb, kept, rp, gamma, beta, rb):
    n, d = x2.shape
    return pl.pallas_call(
        _ln_body,
        grid=(n // rb,),
        in_specs=[
            pl.BlockSpec((rb, d), lambda i: (i, 0)),
            pl.BlockSpec((rb, d), lambda i: (i, 0)),
            pl.BlockSpec((rb, d), lambda i: (i, 0)),
            pl.BlockSpec((rb, 1), lambda i: (i, 0)),
            pl.BlockSpec((rb, 1), lambda i: (i, 0)),
            pl.BlockSpec((1, d), lambda i: (0, 0)),
            pl.BlockSpec((1, d), lambda i: (0, 0)),
        ],
        out_specs=pl.BlockSpec((rb, d), lambda i: (i, 0)),
        out_shape=jax.ShapeDtypeStruct((n, d), jnp.float32),
    )(g, x2, emb, kept, rp, gamma, beta)


def kernel(embed, Wq, bq, Wk, bk, Wv, bv, Wsw, bsw, eW, eb, gamma, beta):
    bdim, s_len, d = embed.shape
    h, dh, _ = Wq.shape
    e_num = Wsw.shape[0]
    n = bdim * s_len
    cap = int(1.2 * n / e_num)
    rb = 256
    cappad = -(-cap // 128) * 128
    trash = e_num * cappad
    scale = float(math.sqrt(d))

    x = embed.reshape(n, d)
    q_all, k_all, v_all = _qkv_call(
        x, Wq.reshape(d, d), Wk.reshape(d, d), Wv.reshape(d, d),
        bq.reshape(1, d), bk.reshape(1, d), bv.reshape(1, d), rb, 256)
    x2 = _attn_call(q_all, k_all, v_all, bdim, h, s_len, dh, 512, scale)

    rp, kept, slot, slotg = _router_call(
        x2, Wsw, bsw.reshape(1, e_num), rb, cap, cappad, trash)
    rp = rp.reshape(n, 1)
    kept = kept.reshape(n, 1)
    slot = slot.reshape(n)
    slotg = slotg.reshape(n)

    inv = jnp.zeros((e_num * cappad + 128,), jnp.int32).at[slot].set(
        jnp.arange(n, dtype=jnp.int32))
    buf = jnp.take(x2, inv[:e_num * cappad + 128], axis=0)
    eout = _expert_call(buf, eW, eb, cappad, 128, 256)
    g = jnp.take(eout, slotg, axis=0)

    out = _ln_call(g, x2, x, kept, rp, gamma.reshape(1, d),
                   beta.reshape(1, d), rb)
    return out.reshape(bdim, s_len, d)


# speedup vs baseline: 1.1929x; 1.1929x over previous
import functools
import math

import jax
import jax.numpy as jnp
from jax import lax
from jax.experimental import pallas as pl
from jax.experimental.pallas import tpu as pltpu
from jax.experimental.pallas import tpu_sc as plsc


def _qkv_body(x_ref, wq_ref, wk_ref, wv_ref, bq_ref, bk_ref, bv_ref,
              q_ref, k_ref, v_ref):
    x = x_ref[...]
    dims = (((1,), (1,)), ((), ()))
    q_ref[...] = lax.dot_general(x, wq_ref[...], dims,
                                 preferred_element_type=jnp.float32) + bq_ref[...]
    k_ref[...] = lax.dot_general(x, wk_ref[...], dims,
                                 preferred_element_type=jnp.float32) + bk_ref[...]
    v_ref[...] = lax.dot_general(x, wv_ref[...], dims,
                                 preferred_element_type=jnp.float32) + bv_ref[...]


def _qkv_call(x, wq, wk, wv, bq, bk, bv, rb, cb):
    n, d = x.shape
    w_spec = pl.BlockSpec((cb, d), lambda j, i: (j, 0))
    b_spec = pl.BlockSpec((1, cb), lambda j, i: (0, j))
    o_spec = pl.BlockSpec((rb, cb), lambda j, i: (i, j))
    o_shape = jax.ShapeDtypeStruct((n, d), jnp.float32)
    return pl.pallas_call(
        _qkv_body,
        grid=(d // cb, n // rb),
        in_specs=[
            pl.BlockSpec((rb, d), lambda j, i: (i, 0)),
            w_spec, w_spec, w_spec, b_spec, b_spec, b_spec,
        ],
        out_specs=[o_spec, o_spec, o_spec],
        out_shape=[o_shape, o_shape, o_shape],
    )(x, wq, wk, wv, bq, bk, bv)


def _attn_body(q_ref, k_ref, v_ref, o_ref, vprev_ref, *, rb, s_len, dh, scale):
    sb = pl.program_id(2)
    q = q_ref[...]
    qa, qb = q[:, :dh], q[:, dh:]
    inv = 1.0 / scale
    dims = (((1,), (1,)), ((), ()))

    @pl.when(sb == 0)
    def _():
        vprev_ref[...] = jnp.zeros_like(vprev_ref)

    def blk(j, mask):
        k2 = k_ref[pl.ds(j * rb, rb), :]
        v2 = v_ref[pl.ds(j * rb, rb), :]
        ea = jnp.exp(lax.dot_general(qa, k2[:, :dh], dims,
                                     preferred_element_type=jnp.float32) * inv)
        eb = jnp.exp(lax.dot_general(qb, k2[:, dh:], dims,
                                     preferred_element_type=jnp.float32) * inv)
        if mask is not None:
            ea = jnp.where(mask, ea, 0.0)
            eb = jnp.where(mask, eb, 0.0)
        return (jnp.dot(ea, v2[:, :dh], preferred_element_type=jnp.float32),
                jnp.sum(ea, axis=-1, keepdims=True),
                jnp.dot(eb, v2[:, dh:], preferred_element_type=jnp.float32),
                jnp.sum(eb, axis=-1, keepdims=True))

    def body(j, carry):
        na, da, nb, db = carry
        xa, ya, xb, yb = blk(j, None)
        return (na + xa, da + ya, nb + xb, db + yb)

    zero_n = jnp.zeros((rb, dh), jnp.float32)
    zero_d = jnp.zeros((rb, 1), jnp.float32)
    na, da, nb, db = lax.fori_loop(0, sb, body,
                                   (zero_n, zero_d, zero_n, zero_d))

    r_i = lax.broadcasted_iota(jnp.int32, (rb, rb), 0)
    c_i = lax.broadcasted_iota(jnp.int32, (rb, rb), 1)
    tril = c_i <= r_i
    xa, ya, xb, yb = blk(sb, tril)
    na, da, nb, db = na + xa, da + ya, nb + xb, db + yb

    vd = v_ref[pl.ds(sb * rb, rb), :]
    vtot = jnp.sum(v_ref[...], axis=0, keepdims=True)
    pref_d = jnp.dot(tril.astype(jnp.float32), vd,
                     preferred_element_type=jnp.float32)
    vprefix = vprev_ref[...] + pref_d
    s_glob = sb * rb + lax.broadcasted_iota(jnp.int32, (rb, 1), 0)
    nfut = (s_len - 1 - s_glob).astype(jnp.float32)
    fut = vtot - vprefix
    vprev_ref[...] = vprev_ref[...] + jnp.sum(vd, axis=0, keepdims=True)

    o_ref[:, :dh] = (na + fut[:, :dh]) / (da + nfut)
    o_ref[:, dh:] = (nb + fut[:, dh:]) / (db + nfut)


def _attn_call(q_all, k_all, v_all, bdim, h, s_len, dh, rb, scale):
    n = q_all.shape[0]
    d = h * dh
    sb_n = s_len // rb
    return pl.pallas_call(
        functools.partial(_attn_body, rb=rb, s_len=s_len, dh=dh, scale=scale),
        grid=(bdim, h // 2, sb_n),
        in_specs=[
            pl.BlockSpec((rb, 2 * dh), lambda b, hh, sb: (b * sb_n + sb, hh)),
            pl.BlockSpec((s_len, 2 * dh), lambda b, hh, sb: (b, hh)),
            pl.BlockSpec((s_len, 2 * dh), lambda b, hh, sb: (b, hh)),
        ],
        out_specs=pl.BlockSpec((rb, 2 * dh),
                               lambda b, hh, sb: (b * sb_n + sb, hh)),
        out_shape=jax.ShapeDtypeStruct((n, d), jnp.float32),
        scratch_shapes=[pltpu.VMEM((1, 2 * dh), jnp.float32)],
    )(q_all, k_all, v_all)


def _router_body(x_ref, w_ref, b_ref, rp_ref, kept_ref, slot_ref, slotg_ref,
                 counts_ref, *, rb, e_num, cap, cappad, trash):
    i = pl.program_id(0)

    @pl.when(i == 0)
    def _():
        counts_ref[...] = jnp.zeros_like(counts_ref)

    logits = lax.dot_general(x_ref[...], w_ref[...], (((1,), (1,)), ((), ())),
                             preferred_element_type=jnp.float32) + b_ref[...]
    m = jnp.max(logits, axis=-1, keepdims=True)
    ex = jnp.exp(logits - m)
    denom = jnp.sum(ex, axis=-1, keepdims=True)
    probs = ex / denom
    rp = 1.0 / denom
    pm = jnp.max(probs, axis=-1, keepdims=True)
    iota_e = lax.broadcasted_iota(jnp.int32, probs.shape, 1)
    route = jnp.min(jnp.where(probs >= pm, iota_e, e_num), axis=-1,
                    keepdims=True)
    onehot = (iota_e == route).astype(jnp.float32)
    r_i = lax.broadcasted_iota(jnp.int32, (rb, rb), 0)
    c_i = lax.broadcasted_iota(jnp.int32, (rb, rb), 1)
    tril = (c_i <= r_i).astype(jnp.float32)
    csum = jnp.dot(tril, onehot, preferred_element_type=jnp.float32)
    rank_all = counts_ref[...] + csum - 1.0
    rank = jnp.sum(rank_all * onehot, axis=-1, keepdims=True)
    counts_ref[...] = counts_ref[...] + csum[rb - 1:rb, :]
    kept = rank < float(cap)
    ranki = rank.astype(jnp.int32)
    slot = route * cappad + ranki
    rp_ref[...] = rp.reshape(1, rb, 1)
    kept_ref[...] = kept.astype(jnp.float32).reshape(1, rb, 1)
    slot_ref[...] = jnp.where(kept, slot, trash).reshape(1, rb, 1)
    slotg_ref[...] = jnp.where(kept, slot, 0).reshape(1, rb, 1)


def _router_call(x2, wsw, bsw, rb, cap, cappad, trash):
    n, d = x2.shape
    e_num = wsw.shape[0]
    nb = n // rb
    outs = pl.pallas_call(
        functools.partial(_router_body, rb=rb, e_num=e_num, cap=cap,
                          cappad=cappad, trash=trash),
        grid=(nb,),
        in_specs=[
            pl.BlockSpec((rb, d), lambda i: (i, 0)),
            pl.BlockSpec((e_num, d), lambda i: (0, 0)),
            pl.BlockSpec((1, e_num), lambda i: (0, 0)),
        ],
        out_specs=[
            pl.BlockSpec((1, rb, 1), lambda i: (i, 0, 0)),
            pl.BlockSpec((1, rb, 1), lambda i: (i, 0, 0)),
            pl.BlockSpec((1, rb, 1), lambda i: (i, 0, 0)),
            pl.BlockSpec((1, rb, 1), lambda i: (i, 0, 0)),
        ],
        out_shape=[
            jax.ShapeDtypeStruct((nb, rb, 1), jnp.float32),
            jax.ShapeDtypeStruct((nb, rb, 1), jnp.float32),
            jax.ShapeDtypeStruct((nb, rb, 1), jnp.int32),
            jax.ShapeDtypeStruct((nb, rb, 1), jnp.int32),
        ],
        scratch_shapes=[pltpu.VMEM((1, e_num), jnp.float32)],
    )(x2, wsw, bsw)
    return outs


_NBUF = 3


def _chunk_pipeline(nch, rd, wr):
    reads = [None] * nch
    writes = [None] * nch
    reads[0] = rd(0)
    for c in range(nch):
        if c + 1 < nch:
            if c + 1 >= _NBUF:
                writes[c + 1 - _NBUF].wait()
            reads[c + 1] = rd(c + 1)
        reads[c].wait()
        writes[c] = wr(c)
    for c in range(max(0, nch - _NBUF), nch):
        writes[c].wait()


def _sc_dispatch(x2, slot, rows_out, d):
    n = x2.shape[0]
    info = plsc.get_sparse_core_info()
    nc, ns = info.num_cores, info.num_subcores
    nw = nc * ns
    tok_w = n // nw
    ch = 16
    nch = tok_w // ch
    mesh = plsc.VectorSubcoreMesh(core_axis_name="c", subcore_axis_name="s")

    @functools.partial(
        pl.kernel, mesh=mesh,
        out_type=jax.ShapeDtypeStruct((rows_out, d), jnp.float32),
        scratch_types=(
            [pltpu.VMEM((ch,), jnp.int32)] * nch
            + [pltpu.VMEM((_NBUF, ch, d), jnp.float32),
               pltpu.SemaphoreType.DMA,
               pltpu.SemaphoreType.DMA]
        ),
    )
    def k(x2_hbm, slot_hbm, buf_hbm, *refs):
        idx_vs = refs[:nch]
        rows_v, sem_r, sem_w = refs[nch:]
        wid = lax.axis_index("s") * nc + lax.axis_index("c")
        base = wid * tok_w
        for c in range(nch):
            pltpu.sync_copy(slot_hbm.at[pl.ds(base + c * ch, ch)], idx_vs[c])

        def rd(c):
            return pltpu.async_copy(
                x2_hbm.at[pl.ds(base + c * ch, ch)],
                rows_v.at[c % _NBUF], sem_r)

        def wr(c):
            return pltpu.async_copy(
                rows_v.at[c % _NBUF], buf_hbm.at[idx_vs[c]], sem_w)

        _chunk_pipeline(nch, rd, wr)

    return k(x2, slot)


def _sc_combine(eout, slotg, d):
    n = slotg.shape[0]
    info = plsc.get_sparse_core_info()
    nc, ns = info.num_cores, info.num_subcores
    nw = nc * ns
    tok_w = n // nw
    ch = 16
    nch = tok_w // ch
    mesh = plsc.VectorSubcoreMesh(core_axis_name="c", subcore_axis_name="s")

    @functools.partial(
        pl.kernel, mesh=mesh,
        out_type=jax.ShapeDtypeStruct((n, d), jnp.float32),
        scratch_types=(
            [pltpu.VMEM((ch,), jnp.int32)] * nch
            + [pltpu.VMEM((_NBUF, ch, d), jnp.float32),
               pltpu.SemaphoreType.DMA,
               pltpu.SemaphoreType.DMA]
        ),
    )
    def k(eout_hbm, slotg_hbm, g_hbm, *refs):
        idx_vs = refs[:nch]
        rows_v, sem_r, sem_w = refs[nch:]
        wid = lax.axis_index("s") * nc + lax.axis_index("c")
        base = wid * tok_w
        for c in range(nch):
            pltpu.sync_copy(slotg_hbm.at[pl.ds(base + c * ch, ch)], idx_vs[c])

        def rd(c):
            return pltpu.async_copy(
                eout_hbm.at[idx_vs[c]], rows_v.at[c % _NBUF], sem_r)

        def wr(c):
            return pltpu.async_copy(
                rows_v.at[c % _NBUF], g_hbm.at[pl.ds(base + c * ch, ch)], sem_w)

        _chunk_pipeline(nch, rd, wr)

    return k(eout, slotg)


def _expert_body(a_ref, w_ref, b_ref, o_ref):
    a_bf = a_ref[...].astype(jnp.bfloat16)
    w_bf = w_ref[0].astype(jnp.bfloat16)
    o_ref[...] = (
        lax.dot_general(a_bf, w_bf, (((1,), (1,)), ((), ())),
                        preferred_element_type=jnp.float32)
        + b_ref[0]
    )


def _expert_call(buf, ew, eb, cappad, rb, cb):
    e_num, d, _ = ew.shape
    ib = cappad // rb
    return pl.pallas_call(
        _expert_body,
        grid=(e_num, d // cb, ib),
        in_specs=[
            pl.BlockSpec((rb, d), lambda e, j, i: (e * ib + i, 0)),
            pl.BlockSpec((1, cb, d), lambda e, j, i: (e, j, 0)),
            pl.BlockSpec((1, 1, cb), lambda e, j, i: (e, 0, j)),
        ],
        out_specs=pl.BlockSpec((rb, cb), lambda e, j, i: (e * ib + i, j)),
        out_shape=jax.ShapeDtypeStruct((e_num * cappad, d), jnp.float32),
    )(buf, ew, eb.reshape(e_num, 1, d))


def _ln_body(g_ref, x2_ref, emb_ref, kept_ref, rp_ref, gam_ref, bet_ref, o_ref):
    kept = kept_ref[...]
    val = g_ref[...] * kept + x2_ref[...] * (1.0 - kept)
    x = val * rp_ref[...] + emb_ref[...]
    mu = jnp.mean(x, axis=-1, keepdims=True)
    xc = x - mu
    var = jnp.mean(xc * xc, axis=-1, keepdims=True)
    o_ref[...] = xc * lax.rsqrt(var + 1e-5) * gam_ref[...] + bet_ref[...]


def _ln_call(g, x2, emb, kept, rp, gamma, beta, rb):
    n, d = x2.shape
    return pl.pallas_call(
        _ln_body,
        grid=(n // rb,),
        in_specs=[
            pl.BlockSpec((rb, d), lambda i: (i, 0)),
            pl.BlockSpec((rb, d), lambda i: (i, 0)),
            pl.BlockSpec((rb, d), lambda i: (i, 0)),
            pl.BlockSpec((rb, 1), lambda i: (i, 0)),
            pl.BlockSpec((rb, 1), lambda i: (i, 0)),
            pl.BlockSpec((1, d), lambda i: (0, 0)),
            pl.BlockSpec((1, d), lambda i: (0, 0)),
        ],
        out_specs=pl.BlockSpec((rb, d), lambda i: (i, 0)),
        out_shape=jax.ShapeDtypeStruct((n, d), jnp.float32),
    )(g, x2, emb, kept, rp, gamma, beta)


def kernel(embed, Wq, bq, Wk, bk, Wv, bv, Wsw, bsw, eW, eb, gamma, beta):
    bdim, s_len, d = embed.shape
    h, dh, _ = Wq.shape
    e_num = Wsw.shape[0]
    n = bdim * s_len
    cap = int(1.2 * n / e_num)
    rb = 256
    cappad = -(-cap // 128) * 128
    trash = e_num * cappad
    scale = float(math.sqrt(d))

    x = embed.reshape(n, d)
    q_all, k_all, v_all = _qkv_call(
        x, Wq.reshape(d, d), Wk.reshape(d, d), Wv.reshape(d, d),
        bq.reshape(1, d), bk.reshape(1, d), bv.reshape(1, d), rb, 256)
    x2 = _attn_call(q_all, k_all, v_all, bdim, h, s_len, dh, 512, scale)

    rp, kept, slot, slotg = _router_call(
        x2, Wsw, bsw.reshape(1, e_num), rb, cap, cappad, trash)
    rp = rp.reshape(n, 1)
    kept = kept.reshape(n, 1)
    slot = slot.reshape(n)
    slotg = slotg.reshape(n)

    buf = _sc_dispatch(x2, slot, e_num * cappad + 128, d)
    eout = _expert_call(buf, eW, eb, cappad, 128, 256)
    g = _sc_combine(eout, slotg, d)

    out = _ln_call(g, x2, x, kept, rp, gamma.reshape(1, d),
                   beta.reshape(1, d), rb)
    return out.reshape(bdim, s_len, d)


# speedup vs baseline: 1.1983x; 1.0045x over previous
import functools
import math

import jax
import jax.numpy as jnp
from jax import lax
from jax.experimental import pallas as pl
from jax.experimental.pallas import tpu as pltpu
from jax.experimental.pallas import tpu_sc as plsc


def _qkv_body(x_ref, wq_ref, wk_ref, wv_ref, bq_ref, bk_ref, bv_ref,
              q_ref, k_ref, v_ref):
    x = x_ref[...]
    dims = (((1,), (1,)), ((), ()))
    q_ref[...] = lax.dot_general(x, wq_ref[...], dims,
                                 preferred_element_type=jnp.float32) + bq_ref[...]
    k_ref[...] = lax.dot_general(x, wk_ref[...], dims,
                                 preferred_element_type=jnp.float32) + bk_ref[...]
    v_ref[...] = lax.dot_general(x, wv_ref[...], dims,
                                 preferred_element_type=jnp.float32) + bv_ref[...]


def _qkv_call(x, wq, wk, wv, bq, bk, bv, rb, cb):
    n, d = x.shape
    w_spec = pl.BlockSpec((cb, d), lambda j, i: (j, 0))
    b_spec = pl.BlockSpec((1, cb), lambda j, i: (0, j))
    o_spec = pl.BlockSpec((rb, cb), lambda j, i: (i, j))
    o_shape = jax.ShapeDtypeStruct((n, d), jnp.float32)
    return pl.pallas_call(
        _qkv_body,
        grid=(d // cb, n // rb),
        in_specs=[
            pl.BlockSpec((rb, d), lambda j, i: (i, 0)),
            w_spec, w_spec, w_spec, b_spec, b_spec, b_spec,
        ],
        out_specs=[o_spec, o_spec, o_spec],
        out_shape=[o_shape, o_shape, o_shape],
    )(x, wq, wk, wv, bq, bk, bv)


def _attn_body(q_ref, k_ref, v_ref, o_ref, vprev_ref, *, rb, s_len, dh, scale):
    sb = pl.program_id(2)
    q = q_ref[...]
    qa, qb = q[:, :dh], q[:, dh:]
    inv = 1.0 / scale
    dims = (((1,), (1,)), ((), ()))

    @pl.when(sb == 0)
    def _():
        vprev_ref[...] = jnp.zeros_like(vprev_ref)

    def blk(j, mask):
        k2 = k_ref[pl.ds(j * rb, rb), :]
        v2 = v_ref[pl.ds(j * rb, rb), :]
        ea = jnp.exp(lax.dot_general(qa, k2[:, :dh], dims,
                                     preferred_element_type=jnp.float32) * inv)
        eb = jnp.exp(lax.dot_general(qb, k2[:, dh:], dims,
                                     preferred_element_type=jnp.float32) * inv)
        if mask is not None:
            ea = jnp.where(mask, ea, 0.0)
            eb = jnp.where(mask, eb, 0.0)
        return (jnp.dot(ea, v2[:, :dh], preferred_element_type=jnp.float32),
                jnp.sum(ea, axis=-1, keepdims=True),
                jnp.dot(eb, v2[:, dh:], preferred_element_type=jnp.float32),
                jnp.sum(eb, axis=-1, keepdims=True))

    def body(j, carry):
        na, da, nb, db = carry
        xa, ya, xb, yb = blk(j, None)
        return (na + xa, da + ya, nb + xb, db + yb)

    zero_n = jnp.zeros((rb, dh), jnp.float32)
    zero_d = jnp.zeros((rb, 1), jnp.float32)
    na, da, nb, db = lax.fori_loop(0, sb, body,
                                   (zero_n, zero_d, zero_n, zero_d))

    r_i = lax.broadcasted_iota(jnp.int32, (rb, rb), 0)
    c_i = lax.broadcasted_iota(jnp.int32, (rb, rb), 1)
    tril = c_i <= r_i
    xa, ya, xb, yb = blk(sb, tril)
    na, da, nb, db = na + xa, da + ya, nb + xb, db + yb

    vd = v_ref[pl.ds(sb * rb, rb), :]
    vtot = jnp.sum(v_ref[...], axis=0, keepdims=True)
    pref_d = jnp.dot(tril.astype(jnp.float32), vd,
                     preferred_element_type=jnp.float32)
    vprefix = vprev_ref[...] + pref_d
    s_glob = sb * rb + lax.broadcasted_iota(jnp.int32, (rb, 1), 0)
    nfut = (s_len - 1 - s_glob).astype(jnp.float32)
    fut = vtot - vprefix
    vprev_ref[...] = vprev_ref[...] + jnp.sum(vd, axis=0, keepdims=True)

    o_ref[:, :dh] = (na + fut[:, :dh]) / (da + nfut)
    o_ref[:, dh:] = (nb + fut[:, dh:]) / (db + nfut)


def _attn_call(q_all, k_all, v_all, bdim, h, s_len, dh, rb, scale):
    n = q_all.shape[0]
    d = h * dh
    sb_n = s_len // rb
    return pl.pallas_call(
        functools.partial(_attn_body, rb=rb, s_len=s_len, dh=dh, scale=scale),
        grid=(bdim, h // 2, sb_n),
        in_specs=[
            pl.BlockSpec((rb, 2 * dh), lambda b, hh, sb: (b * sb_n + sb, hh)),
            pl.BlockSpec((s_len, 2 * dh), lambda b, hh, sb: (b, hh)),
            pl.BlockSpec((s_len, 2 * dh), lambda b, hh, sb: (b, hh)),
        ],
        out_specs=pl.BlockSpec((rb, 2 * dh),
                               lambda b, hh, sb: (b * sb_n + sb, hh)),
        out_shape=jax.ShapeDtypeStruct((n, d), jnp.float32),
        scratch_shapes=[pltpu.VMEM((1, 2 * dh), jnp.float32)],
    )(q_all, k_all, v_all)


def _router_body(x_ref, w_ref, b_ref, rp_ref, kept_ref, slot_ref, slotg_ref,
                 counts_ref, *, rb, e_num, cap, cappad, trash):
    i = pl.program_id(0)

    @pl.when(i == 0)
    def _():
        counts_ref[...] = jnp.zeros_like(counts_ref)

    logits = lax.dot_general(x_ref[...], w_ref[...], (((1,), (1,)), ((), ())),
                             preferred_element_type=jnp.float32) + b_ref[...]
    m = jnp.max(logits, axis=-1, keepdims=True)
    ex = jnp.exp(logits - m)
    denom = jnp.sum(ex, axis=-1, keepdims=True)
    probs = ex / denom
    rp = 1.0 / denom
    pm = jnp.max(probs, axis=-1, keepdims=True)
    iota_e = lax.broadcasted_iota(jnp.int32, probs.shape, 1)
    route = jnp.min(jnp.where(probs >= pm, iota_e, e_num), axis=-1,
                    keepdims=True)
    onehot = (iota_e == route).astype(jnp.float32)
    r_i = lax.broadcasted_iota(jnp.int32, (rb, rb), 0)
    c_i = lax.broadcasted_iota(jnp.int32, (rb, rb), 1)
    tril = (c_i <= r_i).astype(jnp.float32)
    csum = jnp.dot(tril, onehot, preferred_element_type=jnp.float32)
    rank_all = counts_ref[...] + csum - 1.0
    rank = jnp.sum(rank_all * onehot, axis=-1, keepdims=True)
    counts_ref[...] = counts_ref[...] + csum[rb - 1:rb, :]
    kept = rank < float(cap)
    ranki = rank.astype(jnp.int32)
    slot = route * cappad + ranki
    rp_ref[...] = rp.reshape(1, rb, 1)
    kept_ref[...] = kept.astype(jnp.float32).reshape(1, rb, 1)
    slot_ref[...] = jnp.where(kept, slot, trash).reshape(1, rb, 1)
    slotg_ref[...] = jnp.where(kept, slot, 0).reshape(1, rb, 1)


def _router_call(x2, wsw, bsw, rb, cap, cappad, trash):
    n, d = x2.shape
    e_num = wsw.shape[0]
    nb = n // rb
    outs = pl.pallas_call(
        functools.partial(_router_body, rb=rb, e_num=e_num, cap=cap,
                          cappad=cappad, trash=trash),
        grid=(nb,),
        in_specs=[
            pl.BlockSpec((rb, d), lambda i: (i, 0)),
            pl.BlockSpec((e_num, d), lambda i: (0, 0)),
            pl.BlockSpec((1, e_num), lambda i: (0, 0)),
        ],
        out_specs=[
            pl.BlockSpec((1, rb, 1), lambda i: (i, 0, 0)),
            pl.BlockSpec((1, rb, 1), lambda i: (i, 0, 0)),
            pl.BlockSpec((1, rb, 1), lambda i: (i, 0, 0)),
            pl.BlockSpec((1, rb, 1), lambda i: (i, 0, 0)),
        ],
        out_shape=[
            jax.ShapeDtypeStruct((nb, rb, 1), jnp.float32),
            jax.ShapeDtypeStruct((nb, rb, 1), jnp.float32),
            jax.ShapeDtypeStruct((nb, rb, 1), jnp.int32),
            jax.ShapeDtypeStruct((nb, rb, 1), jnp.int32),
        ],
        scratch_shapes=[pltpu.VMEM((1, e_num), jnp.float32)],
    )(x2, wsw, bsw)
    return outs


_NBUF = 3


def _chunk_pipeline(nch, rd, wr):
    reads = [None] * nch
    writes = [None] * nch
    reads[0] = rd(0)
    for c in range(nch):
        if c + 1 < nch:
            if c + 1 >= _NBUF:
                writes[c + 1 - _NBUF].wait()
            reads[c + 1] = rd(c + 1)
        reads[c].wait()
        writes[c] = wr(c)
    for c in range(max(0, nch - _NBUF), nch):
        writes[c].wait()


def _sc_dispatch(x2, slot, rows_out, d):
    n = x2.shape[0]
    info = plsc.get_sparse_core_info()
    nc, ns = info.num_cores, info.num_subcores
    nw = nc * ns
    tok_w = n // nw
    ch = 16
    nch = tok_w // ch
    mesh = plsc.VectorSubcoreMesh(core_axis_name="c", subcore_axis_name="s")

    @functools.partial(
        pl.kernel, mesh=mesh,
        out_type=jax.ShapeDtypeStruct((rows_out, d), jnp.float32),
        scratch_types=(
            [pltpu.VMEM((ch,), jnp.int32)] * nch
            + [pltpu.VMEM((_NBUF, ch, d), jnp.float32),
               pltpu.SemaphoreType.DMA,
               pltpu.SemaphoreType.DMA]
        ),
    )
    def k(x2_hbm, slot_hbm, buf_hbm, *refs):
        idx_vs = refs[:nch]
        rows_v, sem_r, sem_w = refs[nch:]
        wid = lax.axis_index("s") * nc + lax.axis_index("c")
        base = wid * tok_w
        for c in range(nch):
            pltpu.sync_copy(slot_hbm.at[pl.ds(base + c * ch, ch)], idx_vs[c])

        def rd(c):
            return pltpu.async_copy(
                x2_hbm.at[pl.ds(base + c * ch, ch)],
                rows_v.at[c % _NBUF], sem_r)

        def wr(c):
            return pltpu.async_copy(
                rows_v.at[c % _NBUF], buf_hbm.at[idx_vs[c]], sem_w)

        _chunk_pipeline(nch, rd, wr)

    return k(x2, slot)


def _sc_combine(eout, slotg, d):
    n = slotg.shape[0]
    info = plsc.get_sparse_core_info()
    nc, ns = info.num_cores, info.num_subcores
    nw = nc * ns
    tok_w = n // nw
    ch = 16
    nch = tok_w // ch
    mesh = plsc.VectorSubcoreMesh(core_axis_name="c", subcore_axis_name="s")

    @functools.partial(
        pl.kernel, mesh=mesh,
        out_type=jax.ShapeDtypeStruct((n, d), jnp.float32),
        scratch_types=(
            [pltpu.VMEM((ch,), jnp.int32)] * nch
            + [pltpu.VMEM((_NBUF, ch, d), jnp.float32),
               pltpu.SemaphoreType.DMA,
               pltpu.SemaphoreType.DMA]
        ),
    )
    def k(eout_hbm, slotg_hbm, g_hbm, *refs):
        idx_vs = refs[:nch]
        rows_v, sem_r, sem_w = refs[nch:]
        wid = lax.axis_index("s") * nc + lax.axis_index("c")
        base = wid * tok_w
        for c in range(nch):
            pltpu.sync_copy(slotg_hbm.at[pl.ds(base + c * ch, ch)], idx_vs[c])

        def rd(c):
            return pltpu.async_copy(
                eout_hbm.at[idx_vs[c]], rows_v.at[c % _NBUF], sem_r)

        def wr(c):
            return pltpu.async_copy(
                rows_v.at[c % _NBUF], g_hbm.at[pl.ds(base + c * ch, ch)], sem_w)

        _chunk_pipeline(nch, rd, wr)

    return k(eout, slotg)


def _expert_body(a_ref, w_ref, b_ref, o_ref):
    a_bf = a_ref[...].astype(jnp.bfloat16)
    w_bf = w_ref[0].astype(jnp.bfloat16)
    o_ref[...] = (
        lax.dot_general(a_bf, w_bf, (((1,), (1,)), ((), ())),
                        preferred_element_type=jnp.float32)
        + b_ref[0]
    )


def _expert_call(buf, ew, eb, cappad, rb, cb):
    e_num, d, _ = ew.shape
    ib = cappad // rb
    return pl.pallas_call(
        _expert_body,
        grid=(e_num, d // cb, ib),
        in_specs=[
            pl.BlockSpec((rb, d), lambda e, j, i: (e * ib + i, 0)),
            pl.BlockSpec((1, cb, d), lambda e, j, i: (e, j, 0)),
            pl.BlockSpec((1, 1, cb), lambda e, j, i: (e, 0, j)),
        ],
        out_specs=pl.BlockSpec((rb, cb), lambda e, j, i: (e * ib + i, j)),
        out_shape=jax.ShapeDtypeStruct((e_num * cappad, d), jnp.float32),
    )(buf, ew, eb.reshape(e_num, 1, d))


def _ln_body(g_ref, x2_ref, emb_ref, kept_ref, rp_ref, gam_ref, bet_ref, o_ref):
    kept = kept_ref[...]
    val = g_ref[...] * kept + x2_ref[...] * (1.0 - kept)
    x = val * rp_ref[...] + emb_ref[...]
    mu = jnp.mean(x, axis=-1, keepdims=True)
    xc = x - mu
    var = jnp.mean(xc * xc, axis=-1, keepdims=True)
    o_ref[...] = xc * lax.rsqrt(var + 1e-5) * gam_ref[...] + bet_ref[...]


def _ln_call(g, x2, emb, kept, rp, gamma, beta, rb):
    n, d = x2.shape
    return pl.pallas_call(
        _ln_body,
        grid=(n // rb,),
        in_specs=[
            pl.BlockSpec((rb, d), lambda i: (i, 0)),
            pl.BlockSpec((rb, d), lambda i: (i, 0)),
            pl.BlockSpec((rb, d), lambda i: (i, 0)),
            pl.BlockSpec((rb, 1), lambda i: (i, 0)),
            pl.BlockSpec((rb, 1), lambda i: (i, 0)),
            pl.BlockSpec((1, d), lambda i: (0, 0)),
            pl.BlockSpec((1, d), lambda i: (0, 0)),
        ],
        out_specs=pl.BlockSpec((rb, d), lambda i: (i, 0)),
        out_shape=jax.ShapeDtypeStruct((n, d), jnp.float32),
    )(g, x2, emb, kept, rp, gamma, beta)


def kernel(embed, Wq, bq, Wk, bk, Wv, bv, Wsw, bsw, eW, eb, gamma, beta):
    bdim, s_len, d = embed.shape
    h, dh, _ = Wq.shape
    e_num = Wsw.shape[0]
    n = bdim * s_len
    cap = int(1.2 * n / e_num)
    rb = 256
    cappad = -(-cap // 128) * 128
    trash = e_num * cappad
    scale = float(math.sqrt(d))

    x = embed.reshape(n, d)
    q_all, k_all, v_all = _qkv_call(
        x, Wq.reshape(d, d), Wk.reshape(d, d), Wv.reshape(d, d),
        bq.reshape(1, d), bk.reshape(1, d), bv.reshape(1, d), rb, 256)
    x2 = _attn_call(q_all, k_all, v_all, bdim, h, s_len, dh, 1024, scale)

    rp, kept, slot, slotg = _router_call(
        x2, Wsw, bsw.reshape(1, e_num), rb, cap, cappad, trash)
    rp = rp.reshape(n, 1)
    kept = kept.reshape(n, 1)
    slot = slot.reshape(n)
    slotg = slotg.reshape(n)

    buf = _sc_dispatch(x2, slot, e_num * cappad + 128, d)
    eout = _expert_call(buf, eW, eb, cappad, 128, 256)
    g = _sc_combine(eout, slotg, d)

    out = _ln_call(g, x2, x, kept, rp, gamma.reshape(1, d),
                   beta.reshape(1, d), rb)
    return out.reshape(bdim, s_len, d)


# speedup vs baseline: 1.5026x; 1.2540x over previous
import functools
import math

import jax
import jax.numpy as jnp
from jax import lax
from jax.experimental import pallas as pl
from jax.experimental.pallas import tpu as pltpu
from jax.experimental.pallas import tpu_sc as plsc


def _qkv_body(x_ref, wq_ref, wk_ref, wv_ref, bq_ref, bk_ref, bv_ref,
              q_ref, k_ref, v_ref):
    x = x_ref[...]
    dims = (((1,), (1,)), ((), ()))
    q_ref[...] = lax.dot_general(x, wq_ref[...], dims,
                                 preferred_element_type=jnp.float32) + bq_ref[...]
    k_ref[...] = lax.dot_general(x, wk_ref[...], dims,
                                 preferred_element_type=jnp.float32) + bk_ref[...]
    v_ref[...] = lax.dot_general(x, wv_ref[...], dims,
                                 preferred_element_type=jnp.float32) + bv_ref[...]


def _qkv_call(x, wq, wk, wv, bq, bk, bv, rb, cb):
    n, d = x.shape
    w_spec = pl.BlockSpec((cb, d), lambda j, i: (j, 0))
    b_spec = pl.BlockSpec((1, cb), lambda j, i: (0, j))
    o_spec = pl.BlockSpec((rb, cb), lambda j, i: (i, j))
    o_shape = jax.ShapeDtypeStruct((n, d), jnp.float32)
    return pl.pallas_call(
        _qkv_body,
        grid=(d // cb, n // rb),
        in_specs=[
            pl.BlockSpec((rb, d), lambda j, i: (i, 0)),
            w_spec, w_spec, w_spec, b_spec, b_spec, b_spec,
        ],
        out_specs=[o_spec, o_spec, o_spec],
        out_shape=[o_shape, o_shape, o_shape],
    )(x, wq, wk, wv, bq, bk, bv)


def _attn_body(q_ref, k_ref, v_ref, o_ref, vprev_ref, *, rb, s_len, dh, scale):
    sb = pl.program_id(2)
    q = q_ref[...]
    qa, qb = q[:, :dh], q[:, dh:]
    inv = 1.0 / scale
    dims = (((1,), (1,)), ((), ()))

    @pl.when(sb == 0)
    def _():
        vprev_ref[...] = jnp.zeros_like(vprev_ref)

    def blk(j, mask):
        k2 = k_ref[pl.ds(j * rb, rb), :]
        v2 = v_ref[pl.ds(j * rb, rb), :]
        ea = jnp.exp(lax.dot_general(qa, k2[:, :dh], dims,
                                     preferred_element_type=jnp.float32) * inv)
        eb = jnp.exp(lax.dot_general(qb, k2[:, dh:], dims,
                                     preferred_element_type=jnp.float32) * inv)
        if mask is not None:
            ea = jnp.where(mask, ea, 0.0)
            eb = jnp.where(mask, eb, 0.0)
        return (jnp.dot(ea, v2[:, :dh], preferred_element_type=jnp.float32),
                jnp.sum(ea, axis=-1, keepdims=True),
                jnp.dot(eb, v2[:, dh:], preferred_element_type=jnp.float32),
                jnp.sum(eb, axis=-1, keepdims=True))

    def body(j, carry):
        na, da, nb, db = carry
        xa, ya, xb, yb = blk(j, None)
        return (na + xa, da + ya, nb + xb, db + yb)

    zero_n = jnp.zeros((rb, dh), jnp.float32)
    zero_d = jnp.zeros((rb, 1), jnp.float32)
    na, da, nb, db = lax.fori_loop(0, sb, body,
                                   (zero_n, zero_d, zero_n, zero_d))

    r_i = lax.broadcasted_iota(jnp.int32, (rb, rb), 0)
    c_i = lax.broadcasted_iota(jnp.int32, (rb, rb), 1)
    tril = c_i <= r_i
    xa, ya, xb, yb = blk(sb, tril)
    na, da, nb, db = na + xa, da + ya, nb + xb, db + yb

    vd = v_ref[pl.ds(sb * rb, rb), :]
    vtot = jnp.sum(v_ref[...], axis=0, keepdims=True)
    pref_d = jnp.dot(tril.astype(jnp.float32), vd,
                     preferred_element_type=jnp.float32)
    vprefix = vprev_ref[...] + pref_d
    s_glob = sb * rb + lax.broadcasted_iota(jnp.int32, (rb, 1), 0)
    nfut = (s_len - 1 - s_glob).astype(jnp.float32)
    fut = vtot - vprefix
    vprev_ref[...] = vprev_ref[...] + jnp.sum(vd, axis=0, keepdims=True)

    o_ref[:, :dh] = (na + fut[:, :dh]) / (da + nfut)
    o_ref[:, dh:] = (nb + fut[:, dh:]) / (db + nfut)


def _attn_call(q_all, k_all, v_all, bdim, h, s_len, dh, rb, scale):
    n = q_all.shape[0]
    d = h * dh
    sb_n = s_len // rb
    return pl.pallas_call(
        functools.partial(_attn_body, rb=rb, s_len=s_len, dh=dh, scale=scale),
        grid=(bdim, h // 2, sb_n),
        in_specs=[
            pl.BlockSpec((rb, 2 * dh), lambda b, hh, sb: (b * sb_n + sb, hh)),
            pl.BlockSpec((s_len, 2 * dh), lambda b, hh, sb: (b, hh)),
            pl.BlockSpec((s_len, 2 * dh), lambda b, hh, sb: (b, hh)),
        ],
        out_specs=pl.BlockSpec((rb, 2 * dh),
                               lambda b, hh, sb: (b * sb_n + sb, hh)),
        out_shape=jax.ShapeDtypeStruct((n, d), jnp.float32),
        scratch_shapes=[pltpu.VMEM((1, 2 * dh), jnp.float32)],
    )(q_all, k_all, v_all)


def _router_body(x_ref, w_ref, b_ref, rp_ref, kept_ref, slot_ref, slotg_ref,
                 counts_ref, *, rb, e_num, cap, cappad, trash):
    i = pl.program_id(0)

    @pl.when(i == 0)
    def _():
        counts_ref[...] = jnp.zeros_like(counts_ref)

    logits = lax.dot_general(x_ref[...], w_ref[...], (((1,), (1,)), ((), ())),
                             preferred_element_type=jnp.float32) + b_ref[...]
    m = jnp.max(logits, axis=-1, keepdims=True)
    ex = jnp.exp(logits - m)
    denom = jnp.sum(ex, axis=-1, keepdims=True)
    probs = ex / denom
    rp = 1.0 / denom
    pm = jnp.max(probs, axis=-1, keepdims=True)
    iota_e = lax.broadcasted_iota(jnp.int32, probs.shape, 1)
    route = jnp.min(jnp.where(probs >= pm, iota_e, e_num), axis=-1,
                    keepdims=True)
    onehot = (iota_e == route).astype(jnp.float32)
    r_i = lax.broadcasted_iota(jnp.int32, (rb, rb), 0)
    c_i = lax.broadcasted_iota(jnp.int32, (rb, rb), 1)
    tril = (c_i <= r_i).astype(jnp.float32)
    csum = jnp.dot(tril, onehot, preferred_element_type=jnp.float32)
    rank_all = counts_ref[...] + csum - 1.0
    rank = jnp.sum(rank_all * onehot, axis=-1, keepdims=True)
    counts_ref[...] = counts_ref[...] + csum[rb - 1:rb, :]
    kept = rank < float(cap)
    ranki = rank.astype(jnp.int32)
    slot = route * cappad + ranki
    rp_ref[...] = rp.reshape(1, rb, 1)
    kept_ref[...] = kept.astype(jnp.float32).reshape(1, rb, 1)
    slot_ref[...] = jnp.where(kept, slot, trash).reshape(1, rb, 1)
    slotg_ref[...] = jnp.where(kept, slot, 0).reshape(1, rb, 1)


def _router_call(x2, wsw, bsw, rb, cap, cappad, trash):
    n, d = x2.shape
    e_num = wsw.shape[0]
    nb = n // rb
    outs = pl.pallas_call(
        functools.partial(_router_body, rb=rb, e_num=e_num, cap=cap,
                          cappad=cappad, trash=trash),
        grid=(nb,),
        in_specs=[
            pl.BlockSpec((rb, d), lambda i: (i, 0)),
            pl.BlockSpec((e_num, d), lambda i: (0, 0)),
            pl.BlockSpec((1, e_num), lambda i: (0, 0)),
        ],
        out_specs=[
            pl.BlockSpec((1, rb, 1), lambda i: (i, 0, 0)),
            pl.BlockSpec((1, rb, 1), lambda i: (i, 0, 0)),
            pl.BlockSpec((1, rb, 1), lambda i: (i, 0, 0)),
            pl.BlockSpec((1, rb, 1), lambda i: (i, 0, 0)),
        ],
        out_shape=[
            jax.ShapeDtypeStruct((nb, rb, 1), jnp.float32),
            jax.ShapeDtypeStruct((nb, rb, 1), jnp.float32),
            jax.ShapeDtypeStruct((nb, rb, 1), jnp.int32),
            jax.ShapeDtypeStruct((nb, rb, 1), jnp.int32),
        ],
        scratch_shapes=[pltpu.VMEM((1, e_num), jnp.float32)],
    )(x2, wsw, bsw)
    return outs


_NBUF = 3


def _chunk_pipeline(nch, rd, wr):
    reads = [None] * nch
    writes = [None] * nch
    reads[0] = rd(0)
    for c in range(nch):
        if c + 1 < nch:
            if c + 1 >= _NBUF:
                writes[c + 1 - _NBUF].wait()
            reads[c + 1] = rd(c + 1)
        reads[c].wait()
        writes[c] = wr(c)
    for c in range(max(0, nch - _NBUF), nch):
        writes[c].wait()


def _sc_dispatch(x2, slot, rows_out, d):
    n = x2.shape[0]
    info = plsc.get_sparse_core_info()
    nc, ns = info.num_cores, info.num_subcores
    nw = nc * ns
    tok_w = n // nw
    ch = 16
    nch = tok_w // ch
    mesh = plsc.VectorSubcoreMesh(core_axis_name="c", subcore_axis_name="s")

    @functools.partial(
        pl.kernel, mesh=mesh,
        out_type=jax.ShapeDtypeStruct((rows_out, d), jnp.float32),
        scratch_types=(
            [pltpu.VMEM((ch,), jnp.int32)] * nch
            + [pltpu.VMEM((_NBUF, ch, d), jnp.float32),
               pltpu.SemaphoreType.DMA,
               pltpu.SemaphoreType.DMA]
        ),
    )
    def k(x2_hbm, slot_hbm, buf_hbm, *refs):
        idx_vs = refs[:nch]
        rows_v, sem_r, sem_w = refs[nch:]
        wid = lax.axis_index("s") * nc + lax.axis_index("c")
        base = wid * tok_w
        for c in range(nch):
            pltpu.sync_copy(slot_hbm.at[pl.ds(base + c * ch, ch)], idx_vs[c])

        def rd(c):
            return pltpu.async_copy(
                x2_hbm.at[pl.ds(base + c * ch, ch)],
                rows_v.at[c % _NBUF], sem_r)

        def wr(c):
            return pltpu.async_copy(
                rows_v.at[c % _NBUF], buf_hbm.at[idx_vs[c]], sem_w)

        _chunk_pipeline(nch, rd, wr)

    return k(x2, slot)


def _sc_combine(eout, slotg, d):
    n = slotg.shape[0]
    info = plsc.get_sparse_core_info()
    nc, ns = info.num_cores, info.num_subcores
    nw = nc * ns
    tok_w = n // nw
    ch = 16
    nch = tok_w // ch
    mesh = plsc.VectorSubcoreMesh(core_axis_name="c", subcore_axis_name="s")

    @functools.partial(
        pl.kernel, mesh=mesh,
        out_type=jax.ShapeDtypeStruct((n, d), jnp.float32),
        scratch_types=(
            [pltpu.VMEM((ch,), jnp.int32)] * nch
            + [pltpu.VMEM((_NBUF, ch, d), jnp.float32),
               pltpu.SemaphoreType.DMA,
               pltpu.SemaphoreType.DMA]
        ),
    )
    def k(eout_hbm, slotg_hbm, g_hbm, *refs):
        idx_vs = refs[:nch]
        rows_v, sem_r, sem_w = refs[nch:]
        wid = lax.axis_index("s") * nc + lax.axis_index("c")
        base = wid * tok_w
        for c in range(nch):
            pltpu.sync_copy(slotg_hbm.at[pl.ds(base + c * ch, ch)], idx_vs[c])

        def rd(c):
            return pltpu.async_copy(
                eout_hbm.at[idx_vs[c]], rows_v.at[c % _NBUF], sem_r)

        def wr(c):
            return pltpu.async_copy(
                rows_v.at[c % _NBUF], g_hbm.at[pl.ds(base + c * ch, ch)], sem_w)

        _chunk_pipeline(nch, rd, wr)

    return k(eout, slotg)


def _expert_body(a_ref, w_ref, b_ref, o_ref):
    a_bf = a_ref[...].astype(jnp.bfloat16)
    w_bf = w_ref[0].astype(jnp.bfloat16)
    o_ref[...] = (
        lax.dot_general(a_bf, w_bf, (((1,), (1,)), ((), ())),
                        preferred_element_type=jnp.float32)
        + b_ref[0]
    )


def _expert_call(buf, ew, eb, cappad, rb, cb):
    e_num, d, _ = ew.shape
    ib = cappad // rb
    return pl.pallas_call(
        _expert_body,
        grid=(e_num, d // cb, ib),
        in_specs=[
            pl.BlockSpec((rb, d), lambda e, j, i: (e * ib + i, 0)),
            pl.BlockSpec((1, cb, d), lambda e, j, i: (e, j, 0)),
            pl.BlockSpec((1, 1, cb), lambda e, j, i: (e, 0, j)),
        ],
        out_specs=pl.BlockSpec((rb, cb), lambda e, j, i: (e * ib + i, j)),
        out_shape=jax.ShapeDtypeStruct((e_num * cappad, d), jnp.float32),
    )(buf, ew, eb.reshape(e_num, 1, d))


def _ln_body(g_ref, x2_ref, emb_ref, kept_ref, rp_ref, gam_ref, bet_ref, o_ref):
    kept = kept_ref[...]
    val = g_ref[...] * kept + x2_ref[...] * (1.0 - kept)
    x = val * rp_ref[...] + emb_ref[...]
    mu = jnp.mean(x, axis=-1, keepdims=True)
    xc = x - mu
    var = jnp.mean(xc * xc, axis=-1, keepdims=True)
    o_ref[...] = xc * lax.rsqrt(var + 1e-5) * gam_ref[...] + bet_ref[...]


def _ln_call(g, x2, emb, kept, rp, gamma, beta, rb):
    n, d = x2.shape
    return pl.pallas_call(
        _ln_body,
        grid=(n // rb,),
        in_specs=[
            pl.BlockSpec((rb, d), lambda i: (i, 0)),
            pl.BlockSpec((rb, d), lambda i: (i, 0)),
            pl.BlockSpec((rb, d), lambda i: (i, 0)),
            pl.BlockSpec((rb, 1), lambda i: (i, 0)),
            pl.BlockSpec((rb, 1), lambda i: (i, 0)),
            pl.BlockSpec((1, d), lambda i: (0, 0)),
            pl.BlockSpec((1, d), lambda i: (0, 0)),
        ],
        out_specs=pl.BlockSpec((rb, d), lambda i: (i, 0)),
        out_shape=jax.ShapeDtypeStruct((n, d), jnp.float32),
    )(g, x2, emb, kept, rp, gamma, beta)


def kernel(embed, Wq, bq, Wk, bk, Wv, bv, Wsw, bsw, eW, eb, gamma, beta):
    bdim, s_len, d = embed.shape
    h, dh, _ = Wq.shape
    e_num = Wsw.shape[0]
    n = bdim * s_len
    cap = int(1.2 * n / e_num)
    rb = 256
    cappad = -(-cap // 128) * 128
    trash = e_num * cappad
    scale = float(math.sqrt(d))

    x = embed.reshape(n, d)
    q_all, k_all, v_all = _qkv_call(
        x, Wq.reshape(d, d), Wk.reshape(d, d), Wv.reshape(d, d),
        bq.reshape(1, d), bk.reshape(1, d), bv.reshape(1, d), rb, 512)
    x2 = _attn_call(q_all, k_all, v_all, bdim, h, s_len, dh, 1024, scale)

    rp, kept, slot, slotg = _router_call(
        x2, Wsw, bsw.reshape(1, e_num), rb, cap, cappad, trash)
    rp = rp.reshape(n, 1)
    kept = kept.reshape(n, 1)
    slot = slot.reshape(n)
    slotg = slotg.reshape(n)

    buf = _sc_dispatch(x2, slot, e_num * cappad + 128, d)
    eout = _expert_call(buf, eW, eb, cappad, 320, 512)
    g = _sc_combine(eout, slotg, d)

    out = _ln_call(g, x2, x, kept, rp, gamma.reshape(1, d),
                   beta.reshape(1, d), rb)
    return out.reshape(bdim, s_len, d)


# speedup vs baseline: 1.6158x; 1.0754x over previous
import functools
import math

import jax
import jax.numpy as jnp
from jax import lax
from jax.experimental import pallas as pl
from jax.experimental.pallas import tpu as pltpu
from jax.experimental.pallas import tpu_sc as plsc


def _qkv_body(x_ref, wq_ref, wk_ref, wv_ref, bq_ref, bk_ref, bv_ref,
              q_ref, k_ref, v_ref):
    x = x_ref[...]
    dims = (((1,), (1,)), ((), ()))
    q_ref[...] = lax.dot_general(x, wq_ref[...], dims,
                                 preferred_element_type=jnp.float32) + bq_ref[...]
    k_ref[...] = lax.dot_general(x, wk_ref[...], dims,
                                 preferred_element_type=jnp.float32) + bk_ref[...]
    v_ref[...] = lax.dot_general(x, wv_ref[...], dims,
                                 preferred_element_type=jnp.float32) + bv_ref[...]


def _qkv_call(x, wq, wk, wv, bq, bk, bv, rb, cb):
    n, d = x.shape
    w_spec = pl.BlockSpec((cb, d), lambda j, i: (j, 0))
    b_spec = pl.BlockSpec((1, cb), lambda j, i: (0, j))
    o_spec = pl.BlockSpec((rb, cb), lambda j, i: (i, j))
    o_shape = jax.ShapeDtypeStruct((n, d), jnp.float32)
    return pl.pallas_call(
        _qkv_body,
        grid=(d // cb, n // rb),
        in_specs=[
            pl.BlockSpec((rb, d), lambda j, i: (i, 0)),
            w_spec, w_spec, w_spec, b_spec, b_spec, b_spec,
        ],
        out_specs=[o_spec, o_spec, o_spec],
        out_shape=[o_shape, o_shape, o_shape],
    )(x, wq, wk, wv, bq, bk, bv)


def _attn_body(q_ref, k_ref, v_ref, o_ref, vprev_ref, *, rb, s_len, dh, scale):
    sb = pl.program_id(2)
    q = q_ref[...]
    qa, qb = q[:, :dh], q[:, dh:]
    inv = 1.0 / scale
    dims = (((1,), (1,)), ((), ()))

    @pl.when(sb == 0)
    def _():
        vprev_ref[...] = jnp.zeros_like(vprev_ref)

    def blk(j, mask):
        k2 = k_ref[pl.ds(j * rb, rb), :]
        v2 = v_ref[pl.ds(j * rb, rb), :]
        ea = jnp.exp(lax.dot_general(qa, k2[:, :dh], dims,
                                     preferred_element_type=jnp.float32) * inv)
        eb = jnp.exp(lax.dot_general(qb, k2[:, dh:], dims,
                                     preferred_element_type=jnp.float32) * inv)
        if mask is not None:
            ea = jnp.where(mask, ea, 0.0)
            eb = jnp.where(mask, eb, 0.0)
        return (jnp.dot(ea, v2[:, :dh], preferred_element_type=jnp.float32),
                jnp.sum(ea, axis=-1, keepdims=True),
                jnp.dot(eb, v2[:, dh:], preferred_element_type=jnp.float32),
                jnp.sum(eb, axis=-1, keepdims=True))

    def body(j, carry):
        na, da, nb, db = carry
        xa, ya, xb, yb = blk(j, None)
        return (na + xa, da + ya, nb + xb, db + yb)

    zero_n = jnp.zeros((rb, dh), jnp.float32)
    zero_d = jnp.zeros((rb, 1), jnp.float32)
    na, da, nb, db = lax.fori_loop(0, sb, body,
                                   (zero_n, zero_d, zero_n, zero_d))

    r_i = lax.broadcasted_iota(jnp.int32, (rb, rb), 0)
    c_i = lax.broadcasted_iota(jnp.int32, (rb, rb), 1)
    tril = c_i <= r_i
    xa, ya, xb, yb = blk(sb, tril)
    na, da, nb, db = na + xa, da + ya, nb + xb, db + yb

    vd = v_ref[pl.ds(sb * rb, rb), :]
    vtot = jnp.sum(v_ref[...], axis=0, keepdims=True)
    pref_d = jnp.dot(tril.astype(jnp.float32), vd,
                     preferred_element_type=jnp.float32)
    vprefix = vprev_ref[...] + pref_d
    s_glob = sb * rb + lax.broadcasted_iota(jnp.int32, (rb, 1), 0)
    nfut = (s_len - 1 - s_glob).astype(jnp.float32)
    fut = vtot - vprefix
    vprev_ref[...] = vprev_ref[...] + jnp.sum(vd, axis=0, keepdims=True)

    o_ref[:, :dh] = (na + fut[:, :dh]) / (da + nfut)
    o_ref[:, dh:] = (nb + fut[:, dh:]) / (db + nfut)


def _attn_call(q_all, k_all, v_all, bdim, h, s_len, dh, rb, scale):
    n = q_all.shape[0]
    d = h * dh
    sb_n = s_len // rb
    return pl.pallas_call(
        functools.partial(_attn_body, rb=rb, s_len=s_len, dh=dh, scale=scale),
        grid=(bdim, h // 2, sb_n),
        in_specs=[
            pl.BlockSpec((rb, 2 * dh), lambda b, hh, sb: (b * sb_n + sb, hh)),
            pl.BlockSpec((s_len, 2 * dh), lambda b, hh, sb: (b, hh)),
            pl.BlockSpec((s_len, 2 * dh), lambda b, hh, sb: (b, hh)),
        ],
        out_specs=pl.BlockSpec((rb, 2 * dh),
                               lambda b, hh, sb: (b * sb_n + sb, hh)),
        out_shape=jax.ShapeDtypeStruct((n, d), jnp.float32),
        scratch_shapes=[pltpu.VMEM((1, 2 * dh), jnp.float32)],
    )(q_all, k_all, v_all)


def _router_body(x_ref, w_ref, b_ref, rp_ref, kept_ref, slot_ref, slotg_ref,
                 counts_ref, *, rb, e_num, cap, cappad, trash):
    i = pl.program_id(0)

    @pl.when(i == 0)
    def _():
        counts_ref[...] = jnp.zeros_like(counts_ref)

    logits = lax.dot_general(x_ref[...], w_ref[...], (((1,), (1,)), ((), ())),
                             preferred_element_type=jnp.float32) + b_ref[...]
    m = jnp.max(logits, axis=-1, keepdims=True)
    ex = jnp.exp(logits - m)
    denom = jnp.sum(ex, axis=-1, keepdims=True)
    probs = ex / denom
    rp = 1.0 / denom
    pm = jnp.max(probs, axis=-1, keepdims=True)
    iota_e = lax.broadcasted_iota(jnp.int32, probs.shape, 1)
    route = jnp.min(jnp.where(probs >= pm, iota_e, e_num), axis=-1,
                    keepdims=True)
    onehot = (iota_e == route).astype(jnp.float32)
    r_i = lax.broadcasted_iota(jnp.int32, (rb, rb), 0)
    c_i = lax.broadcasted_iota(jnp.int32, (rb, rb), 1)
    tril = (c_i <= r_i).astype(jnp.float32)
    csum = jnp.dot(tril, onehot, preferred_element_type=jnp.float32)
    rank_all = counts_ref[...] + csum - 1.0
    rank = jnp.sum(rank_all * onehot, axis=-1, keepdims=True)
    counts_ref[...] = counts_ref[...] + csum[rb - 1:rb, :]
    kept = rank < float(cap)
    ranki = rank.astype(jnp.int32)
    slot = route * cappad + ranki
    rp_ref[...] = rp.reshape(1, rb, 1)
    kept_ref[...] = kept.astype(jnp.float32).reshape(1, rb, 1)
    slot_ref[...] = jnp.where(kept, slot, trash).reshape(1, rb, 1)
    slotg_ref[...] = jnp.where(kept, slot, 0).reshape(1, rb, 1)


def _router_call(x2, wsw, bsw, rb, cap, cappad, trash):
    n, d = x2.shape
    e_num = wsw.shape[0]
    nb = n // rb
    outs = pl.pallas_call(
        functools.partial(_router_body, rb=rb, e_num=e_num, cap=cap,
                          cappad=cappad, trash=trash),
        grid=(nb,),
        in_specs=[
            pl.BlockSpec((rb, d), lambda i: (i, 0)),
            pl.BlockSpec((e_num, d), lambda i: (0, 0)),
            pl.BlockSpec((1, e_num), lambda i: (0, 0)),
        ],
        out_specs=[
            pl.BlockSpec((1, rb, 1), lambda i: (i, 0, 0)),
            pl.BlockSpec((1, rb, 1), lambda i: (i, 0, 0)),
            pl.BlockSpec((1, rb, 1), lambda i: (i, 0, 0)),
            pl.BlockSpec((1, rb, 1), lambda i: (i, 0, 0)),
        ],
        out_shape=[
            jax.ShapeDtypeStruct((nb, rb, 1), jnp.float32),
            jax.ShapeDtypeStruct((nb, rb, 1), jnp.float32),
            jax.ShapeDtypeStruct((nb, rb, 1), jnp.int32),
            jax.ShapeDtypeStruct((nb, rb, 1), jnp.int32),
        ],
        scratch_shapes=[pltpu.VMEM((1, e_num), jnp.float32)],
    )(x2, wsw, bsw)
    return outs


_NBUF = 3


def _chunk_pipeline(nch, rd, wr):
    reads = [None] * nch
    writes = [None] * nch
    reads[0] = rd(0)
    for c in range(nch):
        if c + 1 < nch:
            if c + 1 >= _NBUF:
                writes[c + 1 - _NBUF].wait()
            reads[c + 1] = rd(c + 1)
        reads[c].wait()
        writes[c] = wr(c)
    for c in range(max(0, nch - _NBUF), nch):
        writes[c].wait()


def _sc_dispatch(x2, slot, rows_out, d):
    n = x2.shape[0]
    info = plsc.get_sparse_core_info()
    nc, ns = info.num_cores, info.num_subcores
    nw = nc * ns
    tok_w = n // nw
    ch = 16
    nch = tok_w // ch
    mesh = plsc.VectorSubcoreMesh(core_axis_name="c", subcore_axis_name="s")

    @functools.partial(
        pl.kernel, mesh=mesh,
        out_type=jax.ShapeDtypeStruct((rows_out, d), jnp.float32),
        scratch_types=(
            [pltpu.VMEM((ch,), jnp.int32)] * nch
            + [pltpu.VMEM((_NBUF, ch, d), jnp.float32),
               pltpu.SemaphoreType.DMA,
               pltpu.SemaphoreType.DMA]
        ),
    )
    def k(x2_hbm, slot_hbm, buf_hbm, *refs):
        idx_vs = refs[:nch]
        rows_v, sem_r, sem_w = refs[nch:]
        wid = lax.axis_index("s") * nc + lax.axis_index("c")
        base = wid * tok_w
        for c in range(nch):
            pltpu.sync_copy(slot_hbm.at[pl.ds(base + c * ch, ch)], idx_vs[c])

        def rd(c):
            return pltpu.async_copy(
                x2_hbm.at[pl.ds(base + c * ch, ch)],
                rows_v.at[c % _NBUF], sem_r)

        def wr(c):
            return pltpu.async_copy(
                rows_v.at[c % _NBUF], buf_hbm.at[idx_vs[c]], sem_w)

        _chunk_pipeline(nch, rd, wr)

    return k(x2, slot)


def _sc_combine(eout, slotg, d):
    n = slotg.shape[0]
    info = plsc.get_sparse_core_info()
    nc, ns = info.num_cores, info.num_subcores
    nw = nc * ns
    tok_w = n // nw
    ch = 16
    nch = tok_w // ch
    mesh = plsc.VectorSubcoreMesh(core_axis_name="c", subcore_axis_name="s")

    @functools.partial(
        pl.kernel, mesh=mesh,
        out_type=jax.ShapeDtypeStruct((n, d), jnp.float32),
        scratch_types=(
            [pltpu.VMEM((ch,), jnp.int32)] * nch
            + [pltpu.VMEM((_NBUF, ch, d), jnp.float32),
               pltpu.SemaphoreType.DMA,
               pltpu.SemaphoreType.DMA]
        ),
    )
    def k(eout_hbm, slotg_hbm, g_hbm, *refs):
        idx_vs = refs[:nch]
        rows_v, sem_r, sem_w = refs[nch:]
        wid = lax.axis_index("s") * nc + lax.axis_index("c")
        base = wid * tok_w
        for c in range(nch):
            pltpu.sync_copy(slotg_hbm.at[pl.ds(base + c * ch, ch)], idx_vs[c])

        def rd(c):
            return pltpu.async_copy(
                eout_hbm.at[idx_vs[c]], rows_v.at[c % _NBUF], sem_r)

        def wr(c):
            return pltpu.async_copy(
                rows_v.at[c % _NBUF], g_hbm.at[pl.ds(base + c * ch, ch)], sem_w)

        _chunk_pipeline(nch, rd, wr)

    return k(eout, slotg)


def _expert_body(a_ref, w_ref, b_ref, o_ref):
    a_bf = a_ref[...].astype(jnp.bfloat16)
    w_bf = w_ref[0].astype(jnp.bfloat16)
    o_ref[...] = (
        lax.dot_general(a_bf, w_bf, (((1,), (1,)), ((), ())),
                        preferred_element_type=jnp.float32)
        + b_ref[0]
    )


def _expert_call(buf, ew, eb, cappad, rb, cb):
    e_num, d, _ = ew.shape
    ib = cappad // rb
    return pl.pallas_call(
        _expert_body,
        grid=(e_num, d // cb, ib),
        in_specs=[
            pl.BlockSpec((rb, d), lambda e, j, i: (e * ib + i, 0)),
            pl.BlockSpec((1, cb, d), lambda e, j, i: (e, j, 0)),
            pl.BlockSpec((1, 1, cb), lambda e, j, i: (e, 0, j)),
        ],
        out_specs=pl.BlockSpec((rb, cb), lambda e, j, i: (e * ib + i, j)),
        out_shape=jax.ShapeDtypeStruct((e_num * cappad, d), jnp.float32),
    )(buf, ew, eb.reshape(e_num, 1, d))


def _ln_body(g_ref, x2_ref, emb_ref, kept_ref, rp_ref, gam_ref, bet_ref, o_ref):
    kept = kept_ref[...]
    val = g_ref[...] * kept + x2_ref[...] * (1.0 - kept)
    x = val * rp_ref[...] + emb_ref[...]
    mu = jnp.mean(x, axis=-1, keepdims=True)
    xc = x - mu
    var = jnp.mean(xc * xc, axis=-1, keepdims=True)
    o_ref[...] = xc * lax.rsqrt(var + 1e-5) * gam_ref[...] + bet_ref[...]


def _ln_call(g, x2, emb, kept, rp, gamma, beta, rb):
    n, d = x2.shape
    return pl.pallas_call(
        _ln_body,
        grid=(n // rb,),
        in_specs=[
            pl.BlockSpec((rb, d), lambda i: (i, 0)),
            pl.BlockSpec((rb, d), lambda i: (i, 0)),
            pl.BlockSpec((rb, d), lambda i: (i, 0)),
            pl.BlockSpec((rb, 1), lambda i: (i, 0)),
            pl.BlockSpec((rb, 1), lambda i: (i, 0)),
            pl.BlockSpec((1, d), lambda i: (0, 0)),
            pl.BlockSpec((1, d), lambda i: (0, 0)),
        ],
        out_specs=pl.BlockSpec((rb, d), lambda i: (i, 0)),
        out_shape=jax.ShapeDtypeStruct((n, d), jnp.float32),
    )(g, x2, emb, kept, rp, gamma, beta)


def kernel(embed, Wq, bq, Wk, bk, Wv, bv, Wsw, bsw, eW, eb, gamma, beta):
    bdim, s_len, d = embed.shape
    h, dh, _ = Wq.shape
    e_num = Wsw.shape[0]
    n = bdim * s_len
    cap = int(1.2 * n / e_num)
    rb = 256
    cappad = -(-cap // 128) * 128
    trash = e_num * cappad
    scale = float(math.sqrt(d))

    x = embed.reshape(n, d)
    q_all, k_all, v_all = _qkv_call(
        x, Wq.reshape(d, d), Wk.reshape(d, d), Wv.reshape(d, d),
        bq.reshape(1, d), bk.reshape(1, d), bv.reshape(1, d), 512, 512)
    x2 = _attn_call(q_all, k_all, v_all, bdim, h, s_len, dh, 1024, scale)

    rp, kept, slot, slotg = _router_call(
        x2, Wsw, bsw.reshape(1, e_num), rb, cap, cappad, trash)
    rp = rp.reshape(n, 1)
    kept = kept.reshape(n, 1)
    slot = slot.reshape(n)
    slotg = slotg.reshape(n)

    buf = _sc_dispatch(x2, slot, e_num * cappad + 128, d)
    eout = _expert_call(buf, eW, eb, cappad, 640, 512)
    g = _sc_combine(eout, slotg, d)

    out = _ln_call(g, x2, x, kept, rp, gamma.reshape(1, d),
                   beta.reshape(1, d), 512)
    return out.reshape(bdim, s_len, d)


# speedup vs baseline: 1.6363x; 1.0127x over previous
import functools
import math

import jax
import jax.numpy as jnp
from jax import lax
from jax.experimental import pallas as pl
from jax.experimental.pallas import tpu as pltpu
from jax.experimental.pallas import tpu_sc as plsc


def _qkv_body(x_ref, wq_ref, wk_ref, wv_ref, bq_ref, bk_ref, bv_ref,
              q_ref, k_ref, v_ref):
    x = x_ref[...]
    dims = (((1,), (1,)), ((), ()))
    q_ref[...] = lax.dot_general(x, wq_ref[...], dims,
                                 preferred_element_type=jnp.float32) + bq_ref[...]
    k_ref[...] = lax.dot_general(x, wk_ref[...], dims,
                                 preferred_element_type=jnp.float32) + bk_ref[...]
    v_ref[...] = lax.dot_general(x, wv_ref[...], dims,
                                 preferred_element_type=jnp.float32) + bv_ref[...]


def _qkv_call(x, wq, wk, wv, bq, bk, bv, rb, cb):
    n, d = x.shape
    w_spec = pl.BlockSpec((cb, d), lambda j, i: (j, 0))
    b_spec = pl.BlockSpec((1, cb), lambda j, i: (0, j))
    o_spec = pl.BlockSpec((rb, cb), lambda j, i: (i, j))
    o_shape = jax.ShapeDtypeStruct((n, d), jnp.float32)
    return pl.pallas_call(
        _qkv_body,
        grid=(d // cb, n // rb),
        in_specs=[
            pl.BlockSpec((rb, d), lambda j, i: (i, 0)),
            w_spec, w_spec, w_spec, b_spec, b_spec, b_spec,
        ],
        out_specs=[o_spec, o_spec, o_spec],
        out_shape=[o_shape, o_shape, o_shape],
    )(x, wq, wk, wv, bq, bk, bv)


def _attn_body(q_ref, k_ref, v_ref, o_ref, vprev_ref, *, rb, s_len, dh, scale):
    sb = pl.program_id(2)
    q = q_ref[...]
    qa, qb = q[:, :dh], q[:, dh:]
    inv = 1.0 / scale
    dims = (((1,), (1,)), ((), ()))

    @pl.when(sb == 0)
    def _():
        vprev_ref[...] = jnp.zeros_like(vprev_ref)

    def blk(j, mask):
        k2 = k_ref[pl.ds(j * rb, rb), :]
        v2 = v_ref[pl.ds(j * rb, rb), :]
        ea = jnp.exp(lax.dot_general(qa, k2[:, :dh], dims,
                                     preferred_element_type=jnp.float32) * inv)
        eb = jnp.exp(lax.dot_general(qb, k2[:, dh:], dims,
                                     preferred_element_type=jnp.float32) * inv)
        if mask is not None:
            ea = jnp.where(mask, ea, 0.0)
            eb = jnp.where(mask, eb, 0.0)
        return (jnp.dot(ea, v2[:, :dh], preferred_element_type=jnp.float32),
                jnp.sum(ea, axis=-1, keepdims=True),
                jnp.dot(eb, v2[:, dh:], preferred_element_type=jnp.float32),
                jnp.sum(eb, axis=-1, keepdims=True))

    def body(j, carry):
        na, da, nb, db = carry
        xa, ya, xb, yb = blk(j, None)
        return (na + xa, da + ya, nb + xb, db + yb)

    zero_n = jnp.zeros((rb, dh), jnp.float32)
    zero_d = jnp.zeros((rb, 1), jnp.float32)
    na, da, nb, db = lax.fori_loop(0, sb, body,
                                   (zero_n, zero_d, zero_n, zero_d))

    r_i = lax.broadcasted_iota(jnp.int32, (rb, rb), 0)
    c_i = lax.broadcasted_iota(jnp.int32, (rb, rb), 1)
    tril = c_i <= r_i
    xa, ya, xb, yb = blk(sb, tril)
    na, da, nb, db = na + xa, da + ya, nb + xb, db + yb

    vd = v_ref[pl.ds(sb * rb, rb), :]
    vtot = jnp.sum(v_ref[...], axis=0, keepdims=True)
    pref_d = jnp.dot(tril.astype(jnp.float32), vd,
                     preferred_element_type=jnp.float32)
    vprefix = vprev_ref[...] + pref_d
    s_glob = sb * rb + lax.broadcasted_iota(jnp.int32, (rb, 1), 0)
    nfut = (s_len - 1 - s_glob).astype(jnp.float32)
    fut = vtot - vprefix
    vprev_ref[...] = vprev_ref[...] + jnp.sum(vd, axis=0, keepdims=True)

    o_ref[:, :dh] = (na + fut[:, :dh]) / (da + nfut)
    o_ref[:, dh:] = (nb + fut[:, dh:]) / (db + nfut)


def _attn_call(q_all, k_all, v_all, bdim, h, s_len, dh, rb, scale):
    n = q_all.shape[0]
    d = h * dh
    sb_n = s_len // rb
    return pl.pallas_call(
        functools.partial(_attn_body, rb=rb, s_len=s_len, dh=dh, scale=scale),
        grid=(bdim, h // 2, sb_n),
        in_specs=[
            pl.BlockSpec((rb, 2 * dh), lambda b, hh, sb: (b * sb_n + sb, hh)),
            pl.BlockSpec((s_len, 2 * dh), lambda b, hh, sb: (b, hh)),
            pl.BlockSpec((s_len, 2 * dh), lambda b, hh, sb: (b, hh)),
        ],
        out_specs=pl.BlockSpec((rb, 2 * dh),
                               lambda b, hh, sb: (b * sb_n + sb, hh)),
        out_shape=jax.ShapeDtypeStruct((n, d), jnp.float32),
        scratch_shapes=[pltpu.VMEM((1, 2 * dh), jnp.float32)],
    )(q_all, k_all, v_all)


def _router_body(x_ref, w_ref, b_ref, rp_ref, kept_ref, slot_ref, slotg_ref,
                 counts_ref, *, rb, e_num, cap, cappad, trash):
    i = pl.program_id(0)

    @pl.when(i == 0)
    def _():
        counts_ref[...] = jnp.zeros_like(counts_ref)

    logits = lax.dot_general(x_ref[...], w_ref[...], (((1,), (1,)), ((), ())),
                             preferred_element_type=jnp.float32) + b_ref[...]
    m = jnp.max(logits, axis=-1, keepdims=True)
    ex = jnp.exp(logits - m)
    denom = jnp.sum(ex, axis=-1, keepdims=True)
    probs = ex / denom
    rp = 1.0 / denom
    pm = jnp.max(probs, axis=-1, keepdims=True)
    iota_e = lax.broadcasted_iota(jnp.int32, probs.shape, 1)
    route = jnp.min(jnp.where(probs >= pm, iota_e, e_num), axis=-1,
                    keepdims=True)
    onehot = (iota_e == route).astype(jnp.float32)
    r_i = lax.broadcasted_iota(jnp.int32, (rb, rb), 0)
    c_i = lax.broadcasted_iota(jnp.int32, (rb, rb), 1)
    tril = (c_i <= r_i).astype(jnp.float32)
    csum = jnp.dot(tril, onehot, preferred_element_type=jnp.float32)
    rank_all = counts_ref[...] + csum - 1.0
    rank = jnp.sum(rank_all * onehot, axis=-1, keepdims=True)
    counts_ref[...] = counts_ref[...] + csum[rb - 1:rb, :]
    kept = rank < float(cap)
    ranki = rank.astype(jnp.int32)
    slot = route * cappad + ranki
    rp_ref[...] = rp.reshape(1, rb, 1)
    kept_ref[...] = kept.astype(jnp.float32).reshape(1, rb, 1)
    slot_ref[...] = jnp.where(kept, slot, trash).reshape(1, rb, 1)
    slotg_ref[...] = jnp.where(kept, slot, 0).reshape(1, rb, 1)


def _router_call(x2, wsw, bsw, rb, cap, cappad, trash):
    n, d = x2.shape
    e_num = wsw.shape[0]
    nb = n // rb
    outs = pl.pallas_call(
        functools.partial(_router_body, rb=rb, e_num=e_num, cap=cap,
                          cappad=cappad, trash=trash),
        grid=(nb,),
        in_specs=[
            pl.BlockSpec((rb, d), lambda i: (i, 0)),
            pl.BlockSpec((e_num, d), lambda i: (0, 0)),
            pl.BlockSpec((1, e_num), lambda i: (0, 0)),
        ],
        out_specs=[
            pl.BlockSpec((1, rb, 1), lambda i: (i, 0, 0)),
            pl.BlockSpec((1, rb, 1), lambda i: (i, 0, 0)),
            pl.BlockSpec((1, rb, 1), lambda i: (i, 0, 0)),
            pl.BlockSpec((1, rb, 1), lambda i: (i, 0, 0)),
        ],
        out_shape=[
            jax.ShapeDtypeStruct((nb, rb, 1), jnp.float32),
            jax.ShapeDtypeStruct((nb, rb, 1), jnp.float32),
            jax.ShapeDtypeStruct((nb, rb, 1), jnp.int32),
            jax.ShapeDtypeStruct((nb, rb, 1), jnp.int32),
        ],
        scratch_shapes=[pltpu.VMEM((1, e_num), jnp.float32)],
    )(x2, wsw, bsw)
    return outs


_NBUF = 3


def _chunk_pipeline(nch, rd, wr):
    reads = [None] * nch
    writes = [None] * nch
    reads[0] = rd(0)
    for c in range(nch):
        if c + 1 < nch:
            if c + 1 >= _NBUF:
                writes[c + 1 - _NBUF].wait()
            reads[c + 1] = rd(c + 1)
        reads[c].wait()
        writes[c] = wr(c)
    for c in range(max(0, nch - _NBUF), nch):
        writes[c].wait()


def _sc_dispatch(x2, slot, rows_out, d):
    n = x2.shape[0]
    info = plsc.get_sparse_core_info()
    nc, ns = info.num_cores, info.num_subcores
    nw = nc * ns
    tok_w = n // nw
    ch = 16
    nch = tok_w // ch
    mesh = plsc.VectorSubcoreMesh(core_axis_name="c", subcore_axis_name="s")

    @functools.partial(
        pl.kernel, mesh=mesh,
        out_type=jax.ShapeDtypeStruct((rows_out, d), jnp.float32),
        scratch_types=(
            [pltpu.VMEM((ch,), jnp.int32)] * nch
            + [pltpu.VMEM((_NBUF, ch, d), jnp.float32),
               pltpu.SemaphoreType.DMA,
               pltpu.SemaphoreType.DMA]
        ),
    )
    def k(x2_hbm, slot_hbm, buf_hbm, *refs):
        idx_vs = refs[:nch]
        rows_v, sem_r, sem_w = refs[nch:]
        wid = lax.axis_index("s") * nc + lax.axis_index("c")
        base = wid * tok_w
        for c in range(nch):
            pltpu.sync_copy(slot_hbm.at[pl.ds(base + c * ch, ch)], idx_vs[c])

        def rd(c):
            return pltpu.async_copy(
                x2_hbm.at[pl.ds(base + c * ch, ch)],
                rows_v.at[c % _NBUF], sem_r)

        def wr(c):
            return pltpu.async_copy(
                rows_v.at[c % _NBUF], buf_hbm.at[idx_vs[c]], sem_w)

        _chunk_pipeline(nch, rd, wr)

    return k(x2, slot)


def _sc_combine(eout, slotg, d):
    n = slotg.shape[0]
    info = plsc.get_sparse_core_info()
    nc, ns = info.num_cores, info.num_subcores
    nw = nc * ns
    tok_w = n // nw
    ch = 16
    nch = tok_w // ch
    mesh = plsc.VectorSubcoreMesh(core_axis_name="c", subcore_axis_name="s")

    @functools.partial(
        pl.kernel, mesh=mesh,
        out_type=jax.ShapeDtypeStruct((n, d), jnp.float32),
        scratch_types=(
            [pltpu.VMEM((ch,), jnp.int32)] * nch
            + [pltpu.VMEM((_NBUF, ch, d), jnp.float32),
               pltpu.SemaphoreType.DMA,
               pltpu.SemaphoreType.DMA]
        ),
    )
    def k(eout_hbm, slotg_hbm, g_hbm, *refs):
        idx_vs = refs[:nch]
        rows_v, sem_r, sem_w = refs[nch:]
        wid = lax.axis_index("s") * nc + lax.axis_index("c")
        base = wid * tok_w
        for c in range(nch):
            pltpu.sync_copy(slotg_hbm.at[pl.ds(base + c * ch, ch)], idx_vs[c])

        def rd(c):
            return pltpu.async_copy(
                eout_hbm.at[idx_vs[c]], rows_v.at[c % _NBUF], sem_r)

        def wr(c):
            return pltpu.async_copy(
                rows_v.at[c % _NBUF], g_hbm.at[pl.ds(base + c * ch, ch)], sem_w)

        _chunk_pipeline(nch, rd, wr)

    return k(eout, slotg)


def _expert_body(a_ref, w_ref, b_ref, o_ref):
    a_bf = a_ref[...].astype(jnp.bfloat16)
    w_bf = w_ref[0].astype(jnp.bfloat16)
    o_ref[...] = (
        lax.dot_general(a_bf, w_bf, (((1,), (1,)), ((), ())),
                        preferred_element_type=jnp.float32)
        + b_ref[0]
    )


def _expert_call(buf, ew, eb, cappad, rb, cb):
    e_num, d, _ = ew.shape
    ib = cappad // rb
    return pl.pallas_call(
        _expert_body,
        grid=(e_num, d // cb, ib),
        in_specs=[
            pl.BlockSpec((rb, d), lambda e, j, i: (e * ib + i, 0)),
            pl.BlockSpec((1, cb, d), lambda e, j, i: (e, j, 0)),
            pl.BlockSpec((1, 1, cb), lambda e, j, i: (e, 0, j)),
        ],
        out_specs=pl.BlockSpec((rb, cb), lambda e, j, i: (e * ib + i, j)),
        out_shape=jax.ShapeDtypeStruct((e_num * cappad, d), jnp.float32),
    )(buf, ew, eb.reshape(e_num, 1, d))


def _ln_body(g_ref, x2_ref, emb_ref, kept_ref, rp_ref, gam_ref, bet_ref, o_ref):
    kept = kept_ref[...]
    val = g_ref[...] * kept + x2_ref[...] * (1.0 - kept)
    x = val * rp_ref[...] + emb_ref[...]
    mu = jnp.mean(x, axis=-1, keepdims=True)
    xc = x - mu
    var = jnp.mean(xc * xc, axis=-1, keepdims=True)
    o_ref[...] = xc * lax.rsqrt(var + 1e-5) * gam_ref[...] + bet_ref[...]


def _ln_call(g, x2, emb, kept, rp, gamma, beta, rb):
    n, d = x2.shape
    return pl.pallas_call(
        _ln_body,
        grid=(n // rb,),
        in_specs=[
            pl.BlockSpec((rb, d), lambda i: (i, 0)),
            pl.BlockSpec((rb, d), lambda i: (i, 0)),
            pl.BlockSpec((rb, d), lambda i: (i, 0)),
            pl.BlockSpec((rb, 1), lambda i: (i, 0)),
            pl.BlockSpec((rb, 1), lambda i: (i, 0)),
            pl.BlockSpec((1, d), lambda i: (0, 0)),
            pl.BlockSpec((1, d), lambda i: (0, 0)),
        ],
        out_specs=pl.BlockSpec((rb, d), lambda i: (i, 0)),
        out_shape=jax.ShapeDtypeStruct((n, d), jnp.float32),
    )(g, x2, emb, kept, rp, gamma, beta)


def kernel(embed, Wq, bq, Wk, bk, Wv, bv, Wsw, bsw, eW, eb, gamma, beta):
    bdim, s_len, d = embed.shape
    h, dh, _ = Wq.shape
    e_num = Wsw.shape[0]
    n = bdim * s_len
    cap = int(1.2 * n / e_num)
    rb = 256
    cappad = -(-cap // 128) * 128
    trash = e_num * cappad
    scale = float(math.sqrt(d))

    x = embed.reshape(n, d)
    q_all, k_all, v_all = _qkv_call(
        x, Wq.reshape(d, d), Wk.reshape(d, d), Wv.reshape(d, d),
        bq.reshape(1, d), bk.reshape(1, d), bv.reshape(1, d), 1024, 512)
    x2 = _attn_call(q_all, k_all, v_all, bdim, h, s_len, dh, 1024, scale)

    rp, kept, slot, slotg = _router_call(
        x2, Wsw, bsw.reshape(1, e_num), rb, cap, cappad, trash)
    rp = rp.reshape(n, 1)
    kept = kept.reshape(n, 1)
    slot = slot.reshape(n)
    slotg = slotg.reshape(n)

    buf = _sc_dispatch(x2, slot, e_num * cappad + 128, d)
    eout = _expert_call(buf, eW, eb, cappad, 640, 1024)
    g = _sc_combine(eout, slotg, d)

    out = _ln_call(g, x2, x, kept, rp, gamma.reshape(1, d),
                   beta.reshape(1, d), 512)
    return out.reshape(bdim, s_len, d)


# speedup vs baseline: 1.6385x; 1.0014x over previous
import functools
import math

import jax
import jax.numpy as jnp
from jax import lax
from jax.experimental import pallas as pl
from jax.experimental.pallas import tpu as pltpu
from jax.experimental.pallas import tpu_sc as plsc


def _qkv_body(x_ref, wq_ref, wk_ref, wv_ref, bq_ref, bk_ref, bv_ref,
              q_ref, k_ref, v_ref):
    x = x_ref[...]
    dims = (((1,), (1,)), ((), ()))
    q_ref[...] = lax.dot_general(x, wq_ref[...], dims,
                                 preferred_element_type=jnp.float32) + bq_ref[...]
    k_ref[...] = lax.dot_general(x, wk_ref[...], dims,
                                 preferred_element_type=jnp.float32) + bk_ref[...]
    v_ref[...] = lax.dot_general(x, wv_ref[...], dims,
                                 preferred_element_type=jnp.float32) + bv_ref[...]


def _qkv_call(x, wq, wk, wv, bq, bk, bv, rb, cb):
    n, d = x.shape
    w_spec = pl.BlockSpec((cb, d), lambda j, i: (j, 0))
    b_spec = pl.BlockSpec((1, cb), lambda j, i: (0, j))
    o_spec = pl.BlockSpec((rb, cb), lambda j, i: (i, j))
    o_shape = jax.ShapeDtypeStruct((n, d), jnp.float32)
    return pl.pallas_call(
        _qkv_body,
        grid=(d // cb, n // rb),
        in_specs=[
            pl.BlockSpec((rb, d), lambda j, i: (i, 0)),
            w_spec, w_spec, w_spec, b_spec, b_spec, b_spec,
        ],
        out_specs=[o_spec, o_spec, o_spec],
        out_shape=[o_shape, o_shape, o_shape],
    )(x, wq, wk, wv, bq, bk, bv)


def _attn_body(q_ref, k_ref, v_ref, o_ref, vprev_ref, *, rb, s_len, dh, scale):
    sb = pl.program_id(2)
    q = q_ref[...]
    nh = q.shape[1] // dh
    qs = [q[:, t * dh:(t + 1) * dh] for t in range(nh)]
    inv = 1.0 / scale
    dims = (((1,), (1,)), ((), ()))

    @pl.when(sb == 0)
    def _():
        vprev_ref[...] = jnp.zeros_like(vprev_ref)

    def blk(j, mask):
        k2 = k_ref[pl.ds(j * rb, rb), :]
        v2 = v_ref[pl.ds(j * rb, rb), :]
        out = []
        for t in range(nh):
            e = jnp.exp(lax.dot_general(
                qs[t], k2[:, t * dh:(t + 1) * dh], dims,
                preferred_element_type=jnp.float32) * inv)
            if mask is not None:
                e = jnp.where(mask, e, 0.0)
            out.append(jnp.dot(e, v2[:, t * dh:(t + 1) * dh],
                               preferred_element_type=jnp.float32))
            out.append(jnp.sum(e, axis=-1, keepdims=True))
        return tuple(out)

    def body(j, carry):
        upd = blk(j, None)
        return tuple(c + u for c, u in zip(carry, upd))

    zero_n = jnp.zeros((rb, dh), jnp.float32)
    zero_d = jnp.zeros((rb, 1), jnp.float32)
    acc = lax.fori_loop(0, sb, body, (zero_n, zero_d) * nh)

    r_i = lax.broadcasted_iota(jnp.int32, (rb, rb), 0)
    c_i = lax.broadcasted_iota(jnp.int32, (rb, rb), 1)
    tril = c_i <= r_i
    acc = tuple(c + u for c, u in zip(acc, blk(sb, tril)))

    vd = v_ref[pl.ds(sb * rb, rb), :]
    vtot = jnp.sum(v_ref[...], axis=0, keepdims=True)
    pref_d = jnp.dot(tril.astype(jnp.float32), vd,
                     preferred_element_type=jnp.float32)
    vprefix = vprev_ref[...] + pref_d
    s_glob = sb * rb + lax.broadcasted_iota(jnp.int32, (rb, 1), 0)
    nfut = (s_len - 1 - s_glob).astype(jnp.float32)
    fut = vtot - vprefix
    vprev_ref[...] = vprev_ref[...] + jnp.sum(vd, axis=0, keepdims=True)

    for t in range(nh):
        o_ref[:, t * dh:(t + 1) * dh] = (
            (acc[2 * t] + fut[:, t * dh:(t + 1) * dh]) / (acc[2 * t + 1] + nfut))


def _attn_call(q_all, k_all, v_all, bdim, h, s_len, dh, rb, nh, scale):
    n = q_all.shape[0]
    d = h * dh
    sb_n = s_len // rb
    hw = nh * dh
    return pl.pallas_call(
        functools.partial(_attn_body, rb=rb, s_len=s_len, dh=dh, scale=scale),
        grid=(bdim, h // nh, sb_n),
        in_specs=[
            pl.BlockSpec((rb, hw), lambda b, hh, sb: (b * sb_n + sb, hh)),
            pl.BlockSpec((s_len, hw), lambda b, hh, sb: (b, hh)),
            pl.BlockSpec((s_len, hw), lambda b, hh, sb: (b, hh)),
        ],
        out_specs=pl.BlockSpec((rb, hw),
                               lambda b, hh, sb: (b * sb_n + sb, hh)),
        out_shape=jax.ShapeDtypeStruct((n, d), jnp.float32),
        scratch_shapes=[pltpu.VMEM((1, hw), jnp.float32)],
    )(q_all, k_all, v_all)


def _router_body(x_ref, w_ref, b_ref, rp_ref, kept_ref, slot_ref, slotg_ref,
                 counts_ref, *, rb, e_num, cap, cappad, trash):
    i = pl.program_id(0)

    @pl.when(i == 0)
    def _():
        counts_ref[...] = jnp.zeros_like(counts_ref)

    logits = lax.dot_general(x_ref[...], w_ref[...], (((1,), (1,)), ((), ())),
                             preferred_element_type=jnp.float32) + b_ref[...]
    m = jnp.max(logits, axis=-1, keepdims=True)
    ex = jnp.exp(logits - m)
    denom = jnp.sum(ex, axis=-1, keepdims=True)
    probs = ex / denom
    rp = 1.0 / denom
    pm = jnp.max(probs, axis=-1, keepdims=True)
    iota_e = lax.broadcasted_iota(jnp.int32, probs.shape, 1)
    route = jnp.min(jnp.where(probs >= pm, iota_e, e_num), axis=-1,
                    keepdims=True)
    onehot = (iota_e == route).astype(jnp.float32)
    r_i = lax.broadcasted_iota(jnp.int32, (rb, rb), 0)
    c_i = lax.broadcasted_iota(jnp.int32, (rb, rb), 1)
    tril = (c_i <= r_i).astype(jnp.float32)
    csum = jnp.dot(tril, onehot, preferred_element_type=jnp.float32)
    rank_all = counts_ref[...] + csum - 1.0
    rank = jnp.sum(rank_all * onehot, axis=-1, keepdims=True)
    counts_ref[...] = counts_ref[...] + csum[rb - 1:rb, :]
    kept = rank < float(cap)
    ranki = rank.astype(jnp.int32)
    slot = route * cappad + ranki
    rp_ref[...] = rp.reshape(1, rb, 1)
    kept_ref[...] = kept.astype(jnp.float32).reshape(1, rb, 1)
    slot_ref[...] = jnp.where(kept, slot, trash).reshape(1, rb, 1)
    slotg_ref[...] = jnp.where(kept, slot, 0).reshape(1, rb, 1)


def _router_call(x2, wsw, bsw, rb, cap, cappad, trash):
    n, d = x2.shape
    e_num = wsw.shape[0]
    nb = n // rb
    outs = pl.pallas_call(
        functools.partial(_router_body, rb=rb, e_num=e_num, cap=cap,
                          cappad=cappad, trash=trash),
        grid=(nb,),
        in_specs=[
            pl.BlockSpec((rb, d), lambda i: (i, 0)),
            pl.BlockSpec((e_num, d), lambda i: (0, 0)),
            pl.BlockSpec((1, e_num), lambda i: (0, 0)),
        ],
        out_specs=[
            pl.BlockSpec((1, rb, 1), lambda i: (i, 0, 0)),
            pl.BlockSpec((1, rb, 1), lambda i: (i, 0, 0)),
            pl.BlockSpec((1, rb, 1), lambda i: (i, 0, 0)),
            pl.BlockSpec((1, rb, 1), lambda i: (i, 0, 0)),
        ],
        out_shape=[
            jax.ShapeDtypeStruct((nb, rb, 1), jnp.float32),
            jax.ShapeDtypeStruct((nb, rb, 1), jnp.float32),
            jax.ShapeDtypeStruct((nb, rb, 1), jnp.int32),
            jax.ShapeDtypeStruct((nb, rb, 1), jnp.int32),
        ],
        scratch_shapes=[pltpu.VMEM((1, e_num), jnp.float32)],
    )(x2, wsw, bsw)
    return outs


_NBUF = 3


def _chunk_pipeline(nch, rd, wr):
    reads = [None] * nch
    writes = [None] * nch
    reads[0] = rd(0)
    for c in range(nch):
        if c + 1 < nch:
            if c + 1 >= _NBUF:
                writes[c + 1 - _NBUF].wait()
            reads[c + 1] = rd(c + 1)
        reads[c].wait()
        writes[c] = wr(c)
    for c in range(max(0, nch - _NBUF), nch):
        writes[c].wait()


def _sc_dispatch(x2, slot, rows_out, d):
    n = x2.shape[0]
    info = plsc.get_sparse_core_info()
    nc, ns = info.num_cores, info.num_subcores
    nw = nc * ns
    tok_w = n // nw
    ch = 16
    nch = tok_w // ch
    mesh = plsc.VectorSubcoreMesh(core_axis_name="c", subcore_axis_name="s")

    @functools.partial(
        pl.kernel, mesh=mesh,
        out_type=jax.ShapeDtypeStruct((rows_out, d), jnp.float32),
        scratch_types=(
            [pltpu.VMEM((ch,), jnp.int32)] * nch
            + [pltpu.VMEM((_NBUF, ch, d), jnp.float32),
               pltpu.SemaphoreType.DMA,
               pltpu.SemaphoreType.DMA]
        ),
    )
    def k(x2_hbm, slot_hbm, buf_hbm, *refs):
        idx_vs = refs[:nch]
        rows_v, sem_r, sem_w = refs[nch:]
        wid = lax.axis_index("s") * nc + lax.axis_index("c")
        base = wid * tok_w
        for c in range(nch):
            pltpu.sync_copy(slot_hbm.at[pl.ds(base + c * ch, ch)], idx_vs[c])

        def rd(c):
            return pltpu.async_copy(
                x2_hbm.at[pl.ds(base + c * ch, ch)],
                rows_v.at[c % _NBUF], sem_r)

        def wr(c):
            return pltpu.async_copy(
                rows_v.at[c % _NBUF], buf_hbm.at[idx_vs[c]], sem_w)

        _chunk_pipeline(nch, rd, wr)

    return k(x2, slot)


def _sc_combine(eout, slotg, d):
    n = slotg.shape[0]
    info = plsc.get_sparse_core_info()
    nc, ns = info.num_cores, info.num_subcores
    nw = nc * ns
    tok_w = n // nw
    ch = 16
    nch = tok_w // ch
    mesh = plsc.VectorSubcoreMesh(core_axis_name="c", subcore_axis_name="s")

    @functools.partial(
        pl.kernel, mesh=mesh,
        out_type=jax.ShapeDtypeStruct((n, d), jnp.float32),
        scratch_types=(
            [pltpu.VMEM((ch,), jnp.int32)] * nch
            + [pltpu.VMEM((_NBUF, ch, d), jnp.float32),
               pltpu.SemaphoreType.DMA,
               pltpu.SemaphoreType.DMA]
        ),
    )
    def k(eout_hbm, slotg_hbm, g_hbm, *refs):
        idx_vs = refs[:nch]
        rows_v, sem_r, sem_w = refs[nch:]
        wid = lax.axis_index("s") * nc + lax.axis_index("c")
        base = wid * tok_w
        for c in range(nch):
            pltpu.sync_copy(slotg_hbm.at[pl.ds(base + c * ch, ch)], idx_vs[c])

        def rd(c):
            return pltpu.async_copy(
                eout_hbm.at[idx_vs[c]], rows_v.at[c % _NBUF], sem_r)

        def wr(c):
            return pltpu.async_copy(
                rows_v.at[c % _NBUF], g_hbm.at[pl.ds(base + c * ch, ch)], sem_w)

        _chunk_pipeline(nch, rd, wr)

    return k(eout, slotg)


def _expert_body(a_ref, w_ref, b_ref, o_ref):
    a_bf = a_ref[...].astype(jnp.bfloat16)
    w_bf = w_ref[0].astype(jnp.bfloat16)
    o_ref[...] = (
        lax.dot_general(a_bf, w_bf, (((1,), (1,)), ((), ())),
                        preferred_element_type=jnp.float32)
        + b_ref[0]
    )


def _expert_call(buf, ew, eb, cappad, rb, cb):
    e_num, d, _ = ew.shape
    ib = cappad // rb
    return pl.pallas_call(
        _expert_body,
        grid=(e_num, d // cb, ib),
        in_specs=[
            pl.BlockSpec((rb, d), lambda e, j, i: (e * ib + i, 0)),
            pl.BlockSpec((1, cb, d), lambda e, j, i: (e, j, 0)),
            pl.BlockSpec((1, 1, cb), lambda e, j, i: (e, 0, j)),
        ],
        out_specs=pl.BlockSpec((rb, cb), lambda e, j, i: (e * ib + i, j)),
        out_shape=jax.ShapeDtypeStruct((e_num * cappad, d), jnp.float32),
    )(buf, ew, eb.reshape(e_num, 1, d))


def _ln_body(g_ref, x2_ref, emb_ref, kept_ref, rp_ref, gam_ref, bet_ref, o_ref):
    kept = kept_ref[...]
    val = g_ref[...] * kept + x2_ref[...] * (1.0 - kept)
    x = val * rp_ref[...] + emb_ref[...]
    mu = jnp.mean(x, axis=-1, keepdims=True)
    xc = x - mu
    var = jnp.mean(xc * xc, axis=-1, keepdims=True)
    o_ref[...] = xc * lax.rsqrt(var + 1e-5) * gam_ref[...] + bet_ref[...]


def _ln_call(g, x2, emb, kept, rp, gamma, beta, rb):
    n, d = x2.shape
    return pl.pallas_call(
        _ln_body,
        grid=(n // rb,),
        in_specs=[
            pl.BlockSpec((rb, d), lambda i: (i, 0)),
            pl.BlockSpec((rb, d), lambda i: (i, 0)),
            pl.BlockSpec((rb, d), lambda i: (i, 0)),
            pl.BlockSpec((rb, 1), lambda i: (i, 0)),
            pl.BlockSpec((rb, 1), lambda i: (i, 0)),
            pl.BlockSpec((1, d), lambda i: (0, 0)),
            pl.BlockSpec((1, d), lambda i: (0, 0)),
        ],
        out_specs=pl.BlockSpec((rb, d), lambda i: (i, 0)),
        out_shape=jax.ShapeDtypeStruct((n, d), jnp.float32),
    )(g, x2, emb, kept, rp, gamma, beta)


def kernel(embed, Wq, bq, Wk, bk, Wv, bv, Wsw, bsw, eW, eb, gamma, beta):
    bdim, s_len, d = embed.shape
    h, dh, _ = Wq.shape
    e_num = Wsw.shape[0]
    n = bdim * s_len
    cap = int(1.2 * n / e_num)
    rb = 256
    cappad = -(-cap // 128) * 128
    trash = e_num * cappad
    scale = float(math.sqrt(d))

    x = embed.reshape(n, d)
    q_all, k_all, v_all = _qkv_call(
        x, Wq.reshape(d, d), Wk.reshape(d, d), Wv.reshape(d, d),
        bq.reshape(1, d), bk.reshape(1, d), bv.reshape(1, d), 1024, 512)
    x2 = _attn_call(q_all, k_all, v_all, bdim, h, s_len, dh, 1024, 4, scale)

    rp, kept, slot, slotg = _router_call(
        x2, Wsw, bsw.reshape(1, e_num), rb, cap, cappad, trash)
    rp = rp.reshape(n, 1)
    kept = kept.reshape(n, 1)
    slot = slot.reshape(n)
    slotg = slotg.reshape(n)

    buf = _sc_dispatch(x2, slot, e_num * cappad + 128, d)
    eout = _expert_call(buf, eW, eb, cappad, 640, 1024)
    g = _sc_combine(eout, slotg, d)

    out = _ln_call(g, x2, x, kept, rp, gamma.reshape(1, d),
                   beta.reshape(1, d), 512)
    return out.reshape(bdim, s_len, d)
